# Initial kernel scaffold; baseline (speedup 1.0000x reference)
import jax
import jax.numpy as jnp
from jax.experimental import pallas as pl


def kernel(x, edge_index, edge_type, hierarchy, W_init, b_init, W_rel_bu, W_self_bu, b_bu, W_rel_mod, W_self_mod, b_mod, W_fin, b_fin):
    raise NotImplementedError("write your pallas kernel here")


# speedup vs baseline: 21.8935x; 21.8935x over previous
import functools

import jax
import jax.numpy as jnp
from jax import lax
from jax.experimental import pallas as pl
from jax.experimental.pallas import tpu as pltpu
from jax.experimental.pallas import tpu_sc as plsc

N = 10000
E = 320000
D = 128
R = 4
M = 1024

NC = 2
NS = 16
NW = NC * NS
EW = E // NW
CHUNK = 128
NFULL = EW // CHUNK
TAIL = EW - NFULL * CHUNK
CW = 16
NCHUNK_N = N // CHUNK
NTAIL_N = N - NCHUNK_N * CHUNK


def _mesh():
    return plsc.VectorSubcoreMesh(core_axis_name="c", subcore_axis_name="s")


def _fill_rows(ref, nrows, ncol, val):
    v = jnp.full((16,), val, jnp.float32)

    def body(i, carry):
        for j in range(ncol // 16):
            ref[i, pl.ds(j * 16, 16)] = v
        return carry

    lax.fori_loop(0, nrows, body, 0)


def _build_prep():
    out_type = (
        jax.ShapeDtypeStruct((E,), jnp.int32),
        jax.ShapeDtypeStruct((E,), jnp.int32),
        jax.ShapeDtypeStruct((E,), jnp.int32),
        jax.ShapeDtypeStruct((NC * N, CW), jnp.float32),
        jax.ShapeDtypeStruct((NC * M, CW), jnp.float32),
        jax.ShapeDtypeStruct((NC * M, CW), jnp.float32),
    )
    scratch = [
        pltpu.VMEM((CHUNK,), jnp.int32),
        pltpu.VMEM((TAIL,), jnp.int32),
        pltpu.VMEM((CHUNK,), jnp.int32),
        pltpu.VMEM((CHUNK,), jnp.int32),
        pltpu.VMEM((CHUNK,), jnp.int32),
        pltpu.VMEM((CHUNK,), jnp.int32),
        pltpu.VMEM((CHUNK,), jnp.int32),
        pltpu.VMEM((CHUNK,), jnp.int32),
        pltpu.VMEM((TAIL,), jnp.int32),
        pltpu.VMEM((TAIL,), jnp.int32),
        pltpu.VMEM((TAIL,), jnp.int32),
        pltpu.VMEM((TAIL,), jnp.int32),
        pltpu.VMEM((TAIL,), jnp.int32),
        pltpu.VMEM((TAIL,), jnp.int32),
        pltpu.VMEM((CHUNK,), jnp.int32),
        pltpu.VMEM((NTAIL_N,), jnp.int32),
        pltpu.VMEM((CHUNK, CW), jnp.float32),
        pltpu.VMEM((80, CW), jnp.float32),
        pltpu.VMEM((M // NS, CW), jnp.float32),
        pltpu.VMEM_SHARED((N, CW), jnp.float32),
        pltpu.VMEM_SHARED((M, CW), jnp.float32),
        pltpu.VMEM_SHARED((M, CW), jnp.float32),
        pltpu.SemaphoreType.DMA,
    ]

    @functools.partial(pl.kernel, out_type=out_type, mesh=_mesh(),
                       scratch_types=scratch, name="sc_prep")
    def prep(srcx, dstx, etype, hier,
             gbu_out, gmod_out, md_out, degb_out, degm_out, cnt_out,
             hs_v, hs_t, s_v, d_v, t_v, gbu_v, gmod_v, md_v,
             s_t, d_t, t_t, gbu_t, gmod_t, md_t, hidx_v, hidx_t,
             ones_v, stage_n, stage_m, degb_acc, degm_acc, cnt_acc, sem):
        c = lax.axis_index("c")
        s = lax.axis_index("s")
        w = c * NS + s
        rm = M // NS
        NZB = N // 80

        _fill_rows(stage_n, 80, CW, 0.0)
        for j in range((NZB + NS - 1) // NS):
            zid = s + NS * j

            @pl.when(zid < NZB)
            def _():
                pltpu.sync_copy(stage_n, degb_acc.at[pl.ds(zid * 80, 80)])
        _fill_rows(stage_m, rm, CW, 0.0)
        pltpu.sync_copy(stage_m, degm_acc.at[pl.ds(s * rm, rm)])
        pltpu.sync_copy(stage_m, cnt_acc.at[pl.ds(s * rm, rm)])
        plsc.subcore_barrier()

        _fill_rows(ones_v, CHUNK, CW, 1.0)

        ebase = w * EW

        def echunk(j, carry):
            base = ebase + j * CHUNK
            pltpu.sync_copy(srcx.at[pl.ds(base, CHUNK)], s_v)
            pltpu.sync_copy(dstx.at[pl.ds(base, CHUNK)], d_v)
            pltpu.sync_copy(etype.at[pl.ds(base, CHUNK)], t_v)
            pltpu.async_copy(hier.at[s_v], hs_v, sem).wait()
            pltpu.async_copy(hier.at[d_v], md_v, sem).wait()

            def vec(i, carry2):
                sv = s_v[pl.ds(i * 16, 16)]
                tv = t_v[pl.ds(i * 16, 16)]
                gbu_v[pl.ds(i * 16, 16)] = tv * N + sv
                gmod_v[pl.ds(i * 16, 16)] = tv * M + hs_v[pl.ds(i * 16, 16)]
                return carry2

            lax.fori_loop(0, CHUNK // 16, vec, 0)
            pltpu.sync_copy(gbu_v, gbu_out.at[pl.ds(base, CHUNK)])
            pltpu.sync_copy(gmod_v, gmod_out.at[pl.ds(base, CHUNK)])
            pltpu.sync_copy(md_v, md_out.at[pl.ds(base, CHUNK)])
            pltpu.sync_copy(ones_v, degb_acc.at[d_v], add=True)
            pltpu.sync_copy(ones_v, degm_acc.at[md_v], add=True)
            return carry

        lax.fori_loop(0, NFULL, echunk, 0)

        tb = ebase + NFULL * CHUNK
        pltpu.sync_copy(srcx.at[pl.ds(tb, TAIL)], s_t)
        pltpu.sync_copy(dstx.at[pl.ds(tb, TAIL)], d_t)
        pltpu.sync_copy(etype.at[pl.ds(tb, TAIL)], t_t)
        pltpu.async_copy(hier.at[s_t], hs_t, sem).wait()
        pltpu.async_copy(hier.at[d_t], md_t, sem).wait()
        sv = s_t[...]
        tv = t_t[...]
        gbu_t[...] = tv * N + sv
        gmod_t[...] = tv * M + hs_t[...]
        pltpu.sync_copy(gbu_t, gbu_out.at[pl.ds(tb, TAIL)])
        pltpu.sync_copy(gmod_t, gmod_out.at[pl.ds(tb, TAIL)])
        pltpu.sync_copy(md_t, md_out.at[pl.ds(tb, TAIL)])
        pltpu.sync_copy(ones_v.at[pl.ds(0, TAIL)], degb_acc.at[d_t], add=True)
        pltpu.sync_copy(ones_v.at[pl.ds(0, TAIL)], degm_acc.at[md_t], add=True)

        for j in range((NCHUNK_N + NW - 1) // NW):
            cid = w + NW * j

            @pl.when(cid < NCHUNK_N)
            def _():
                pltpu.sync_copy(hier.at[pl.ds(cid * CHUNK, CHUNK)], hidx_v)
                pltpu.sync_copy(ones_v, cnt_acc.at[hidx_v], add=True)

        @pl.when(w == 0)
        def _():
            pltpu.sync_copy(hier.at[pl.ds(N - NTAIL_N, NTAIL_N)], hidx_t)
            pltpu.sync_copy(ones_v.at[pl.ds(0, NTAIL_N)], cnt_acc.at[hidx_t],
                            add=True)

        plsc.subcore_barrier()

        for j in range((NZB + NS - 1) // NS):
            zid = s + NS * j

            @pl.when(zid < NZB)
            def _():
                pltpu.sync_copy(degb_acc.at[pl.ds(zid * 80, 80)], stage_n)
                pltpu.sync_copy(stage_n,
                                degb_out.at[pl.ds(c * N + zid * 80, 80)])
        pltpu.sync_copy(degm_acc.at[pl.ds(s * rm, rm)], stage_m)
        pltpu.sync_copy(stage_m, degm_out.at[pl.ds(c * M + s * rm, rm)])
        pltpu.sync_copy(cnt_acc.at[pl.ds(s * rm, rm)], stage_m)
        pltpu.sync_copy(stage_m, cnt_out.at[pl.ds(c * M + s * rm, rm)])

    return prep


def _build_agg(NN, VT, ZCH, NZ, name):
    assert ZCH * NZ == NN
    scratch = [
        pltpu.VMEM_SHARED((NN, D), jnp.float32),
        pltpu.VMEM((CHUNK,), jnp.int32),
        pltpu.VMEM((CHUNK,), jnp.int32),
        pltpu.VMEM((CHUNK, D), jnp.float32),
        pltpu.VMEM((TAIL,), jnp.int32),
        pltpu.VMEM((TAIL,), jnp.int32),
        pltpu.VMEM((TAIL, D), jnp.float32),
        pltpu.VMEM((ZCH, D), jnp.float32),
        pltpu.SemaphoreType.DMA,
    ]

    @functools.partial(pl.kernel,
                       out_type=jax.ShapeDtypeStruct((NC * NN, D), jnp.float32),
                       mesh=_mesh(), scratch_types=scratch, name=name)
    def agg(table, gidx, dstx, out,
            acc, gi_v, di_v, rows_v, gi_t, di_t, rows_t, stage, sem):
        c = lax.axis_index("c")
        s = lax.axis_index("s")
        w = c * NS + s

        _fill_rows(stage, ZCH, D, 0.0)
        for j in range((NZ + NS - 1) // NS):
            cid = s + NS * j

            @pl.when(cid < NZ)
            def _():
                pltpu.sync_copy(stage, acc.at[pl.ds(cid * ZCH, ZCH)])
        plsc.subcore_barrier()

        ebase = w * EW

        def chunk(j, carry):
            base = ebase + j * CHUNK
            pltpu.sync_copy(gidx.at[pl.ds(base, CHUNK)], gi_v)
            pltpu.sync_copy(dstx.at[pl.ds(base, CHUNK)], di_v)
            pltpu.async_copy(table.at[gi_v], rows_v, sem).wait()
            pltpu.sync_copy(rows_v, acc.at[di_v], add=True)
            return carry

        lax.fori_loop(0, NFULL, chunk, 0)

        tb = ebase + NFULL * CHUNK
        pltpu.sync_copy(gidx.at[pl.ds(tb, TAIL)], gi_t)
        pltpu.sync_copy(dstx.at[pl.ds(tb, TAIL)], di_t)
        pltpu.async_copy(table.at[gi_t], rows_t, sem).wait()
        pltpu.sync_copy(rows_t, acc.at[di_t], add=True)

        plsc.subcore_barrier()
        for j in range((NZ + NS - 1) // NS):
            cid = s + NS * j

            @pl.when(cid < NZ)
            def _():
                pltpu.sync_copy(acc.at[pl.ds(cid * ZCH, ZCH)], stage)
                pltpu.sync_copy(stage, out.at[pl.ds(c * NN + cid * ZCH, ZCH)])

    return agg


def _build_pool():
    RC = 80
    NRC = N // RC
    ZCH = M // NS
    scratch = [
        pltpu.VMEM_SHARED((M, D), jnp.float32),
        pltpu.VMEM((RC, D), jnp.float32),
        pltpu.VMEM((RC,), jnp.int32),
        pltpu.VMEM((ZCH, D), jnp.float32),
    ]

    @functools.partial(pl.kernel,
                       out_type=jax.ShapeDtypeStruct((NC * M, D), jnp.float32),
                       mesh=_mesh(), scratch_types=scratch, name="sc_pool")
    def pool(h, hier, out, acc, rows_v, hidx_v, stage):
        c = lax.axis_index("c")
        s = lax.axis_index("s")
        w = c * NS + s

        _fill_rows(stage, ZCH, D, 0.0)
        pltpu.sync_copy(stage, acc.at[pl.ds(s * ZCH, ZCH)])
        plsc.subcore_barrier()

        for j in range((NRC + NW - 1) // NW):
            cid = w + NW * j

            @pl.when(cid < NRC)
            def _():
                base = cid * RC
                pltpu.sync_copy(h.at[pl.ds(base, RC)], rows_v)
                pltpu.sync_copy(hier.at[pl.ds(base, RC)], hidx_v)
                pltpu.sync_copy(rows_v, acc.at[hidx_v], add=True)

        plsc.subcore_barrier()
        pltpu.sync_copy(acc.at[pl.ds(s * ZCH, ZCH)], stage)
        pltpu.sync_copy(stage, out.at[pl.ds(c * M + s * ZCH, ZCH)])

    return pool


def _dot(a, b):
    return jnp.dot(a, b, preferred_element_type=jnp.float32)


def _head_body(x_ref, wi_ref, bi_ref, wrel_ref, wself_ref, b_ref,
               hr_ref, sb_ref):
    h = jnp.maximum(_dot(x_ref[...], wi_ref[...]) + bi_ref[...], 0.0)
    for r in range(R):
        hr_ref[r] = _dot(h, wrel_ref[r])
    sb_ref[...] = _dot(h, wself_ref[...]) + b_ref[...]


def _norm_h(p_ref, deg_ref, sbp_ref):
    degs = deg_ref[0, :, 0:1] + deg_ref[1, :, 0:1]
    inv = 1.0 / jnp.maximum(degs, 1.0)
    return jnp.maximum((p_ref[0] + p_ref[1]) * inv + sbp_ref[...], 0.0)


def _mid_body(p_ref, deg_ref, sbp_ref, wrel_ref, wself_ref, b_ref,
              hr_ref, sb_ref):
    h = _norm_h(p_ref, deg_ref, sbp_ref)
    for r in range(R):
        hr_ref[r] = _dot(h, wrel_ref[r])
    sb_ref[...] = _dot(h, wself_ref[...]) + b_ref[...]


def _combine_body(p_ref, deg_ref, sbp_ref, h_ref):
    h_ref[...] = _norm_h(p_ref, deg_ref, sbp_ref)


def _mod0_body(p_ref, cnt_ref, wrel_ref, wself_ref, b_ref, hr_ref, sb_ref):
    cnts = cnt_ref[0, :, 0:1] + cnt_ref[1, :, 0:1]
    pooled = (p_ref[0] + p_ref[1]) * (1.0 / jnp.maximum(cnts, 1.0))
    for r in range(R):
        hr_ref[r] = _dot(pooled, wrel_ref[r])
    sb_ref[...] = _dot(pooled, wself_ref[...]) + b_ref[...]


def _tail_body(p_ref, deg_ref, sbp_ref, wf_ref, bf_ref, out_ref):
    h = _norm_h(p_ref, deg_ref, sbp_ref)
    out_ref[...] = jnp.maximum(_dot(h, wf_ref[...]) + bf_ref[...], 0.0)


_BN = 1000


def _head_call(x, wi, bi, wrel, wself, b):
    nb = N // _BN
    return pl.pallas_call(
        _head_body,
        grid=(nb,),
        in_specs=[
            pl.BlockSpec((_BN, D), lambda i: (i, 0)),
            pl.BlockSpec((D, D), lambda i: (0, 0)),
            pl.BlockSpec((1, D), lambda i: (0, 0)),
            pl.BlockSpec((R, D, D), lambda i: (0, 0, 0)),
            pl.BlockSpec((D, D), lambda i: (0, 0)),
            pl.BlockSpec((1, D), lambda i: (0, 0)),
        ],
        out_specs=[
            pl.BlockSpec((R, _BN, D), lambda i: (0, i, 0)),
            pl.BlockSpec((_BN, D), lambda i: (i, 0)),
        ],
        out_shape=[
            jax.ShapeDtypeStruct((R, N, D), jnp.float32),
            jax.ShapeDtypeStruct((N, D), jnp.float32),
        ],
    )(x, wi, bi.reshape(1, D), wrel, wself, b.reshape(1, D))


def _mid_call(part, deg2, sbp, wrel, wself, b):
    nb = N // _BN
    return pl.pallas_call(
        _mid_body,
        grid=(nb,),
        in_specs=[
            pl.BlockSpec((NC, _BN, D), lambda i: (0, i, 0)),
            pl.BlockSpec((NC, _BN, CW), lambda i: (0, i, 0)),
            pl.BlockSpec((_BN, D), lambda i: (i, 0)),
            pl.BlockSpec((R, D, D), lambda i: (0, 0, 0)),
            pl.BlockSpec((D, D), lambda i: (0, 0)),
            pl.BlockSpec((1, D), lambda i: (0, 0)),
        ],
        out_specs=[
            pl.BlockSpec((R, _BN, D), lambda i: (0, i, 0)),
            pl.BlockSpec((_BN, D), lambda i: (i, 0)),
        ],
        out_shape=[
            jax.ShapeDtypeStruct((R, N, D), jnp.float32),
            jax.ShapeDtypeStruct((N, D), jnp.float32),
        ],
    )(part.reshape(NC, N, D), deg2.reshape(NC, N, CW), sbp,
      wrel, wself, b.reshape(1, D))


def _combine_call(part, deg2, sbp):
    nb = N // _BN
    return pl.pallas_call(
        _combine_body,
        grid=(nb,),
        in_specs=[
            pl.BlockSpec((NC, _BN, D), lambda i: (0, i, 0)),
            pl.BlockSpec((NC, _BN, CW), lambda i: (0, i, 0)),
            pl.BlockSpec((_BN, D), lambda i: (i, 0)),
        ],
        out_specs=pl.BlockSpec((_BN, D), lambda i: (i, 0)),
        out_shape=jax.ShapeDtypeStruct((N, D), jnp.float32),
    )(part.reshape(NC, N, D), deg2.reshape(NC, N, CW), sbp)


def _mod0_call(pool2, cnt2, wrel, wself, b):
    return pl.pallas_call(
        _mod0_body,
        out_shape=[
            jax.ShapeDtypeStruct((R, M, D), jnp.float32),
            jax.ShapeDtypeStruct((M, D), jnp.float32),
        ],
    )(pool2.reshape(NC, M, D), cnt2.reshape(NC, M, CW),
      wrel, wself, b.reshape(1, D))


def _modmid_call(part, deg2, sbp, wrel, wself, b):
    return pl.pallas_call(
        _mid_body,
        out_shape=[
            jax.ShapeDtypeStruct((R, M, D), jnp.float32),
            jax.ShapeDtypeStruct((M, D), jnp.float32),
        ],
    )(part.reshape(NC, M, D), deg2.reshape(NC, M, CW), sbp,
      wrel, wself, b.reshape(1, D))


def _tail_call(part, deg2, sbp, wf, bf):
    return pl.pallas_call(
        _tail_body,
        out_shape=jax.ShapeDtypeStruct((M, D), jnp.float32),
    )(part.reshape(NC, M, D), deg2.reshape(NC, M, CW), sbp,
      wf, bf.reshape(1, D))


_prep = _build_prep()
_agg_n = _build_agg(N, R * N, 80, 125, "sc_agg_n")
_agg_m = _build_agg(M, R * M, 64, 16, "sc_agg_m")
_pool = _build_pool()


def kernel(x, edge_index, edge_type, hierarchy,
           W_init, b_init, W_rel_bu, W_self_bu, b_bu,
           W_rel_mod, W_self_mod, b_mod, W_fin, b_fin):
    srcx = edge_index[0]
    dstx = edge_index[1]

    gbu, gmod, mdst, degb2, degm2, cnt2 = _prep(srcx, dstx, edge_type,
                                                hierarchy)

    hr, sb = _head_call(x, W_init, b_init, W_rel_bu[0], W_self_bu[0], b_bu[0])
    part = _agg_n(hr.reshape(R * N, D), gbu, dstx)
    hr, sb = _mid_call(part, degb2, sb, W_rel_bu[1], W_self_bu[1], b_bu[1])
    part = _agg_n(hr.reshape(R * N, D), gbu, dstx)
    h2 = _combine_call(part, degb2, sb)

    pool2 = _pool(h2, hierarchy)
    hrm, sbm = _mod0_call(pool2, cnt2, W_rel_mod[0], W_self_mod[0], b_mod[0])
    mp = _agg_m(hrm.reshape(R * M, D), gmod, mdst)
    hrm, sbm = _modmid_call(mp, degm2, sbm, W_rel_mod[1], W_self_mod[1],
                            b_mod[1])
    mp = _agg_m(hrm.reshape(R * M, D), gmod, mdst)

    return _tail_call(mp, degm2, sbm, W_fin, b_fin)


# speedup vs baseline: 21.9068x; 1.0006x over previous
import functools

import jax
import jax.numpy as jnp
from jax import lax
from jax.experimental import pallas as pl
from jax.experimental.pallas import tpu as pltpu
from jax.experimental.pallas import tpu_sc as plsc

N = 10000
E = 320000
D = 128
R = 4
M = 1024

NC = 2
NS = 16
NW = NC * NS
EW = E // NW
CHUNK = 128
NFULL = EW // CHUNK
TAIL = EW - NFULL * CHUNK
CW = 16
NCHUNK_N = N // CHUNK
NTAIL_N = N - NCHUNK_N * CHUNK


def _mesh():
    return plsc.VectorSubcoreMesh(core_axis_name="c", subcore_axis_name="s")


def _build_prep():
    out_type = (
        jax.ShapeDtypeStruct((E,), jnp.int32),
        jax.ShapeDtypeStruct((E,), jnp.int32),
        jax.ShapeDtypeStruct((E,), jnp.int32),
        jax.ShapeDtypeStruct((NC * N,), jnp.float32),
        jax.ShapeDtypeStruct((NC * M,), jnp.float32),
        jax.ShapeDtypeStruct((NC * M,), jnp.float32),
    )
    scratch = [
        pltpu.VMEM((CHUNK,), jnp.int32),
        pltpu.VMEM((TAIL,), jnp.int32),
        pltpu.VMEM((CHUNK,), jnp.int32),
        pltpu.VMEM((CHUNK,), jnp.int32),
        pltpu.VMEM((CHUNK,), jnp.int32),
        pltpu.VMEM((CHUNK,), jnp.int32),
        pltpu.VMEM((CHUNK,), jnp.int32),
        pltpu.VMEM((CHUNK,), jnp.int32),
        pltpu.VMEM((TAIL,), jnp.int32),
        pltpu.VMEM((TAIL,), jnp.int32),
        pltpu.VMEM((TAIL,), jnp.int32),
        pltpu.VMEM((TAIL,), jnp.int32),
        pltpu.VMEM((TAIL,), jnp.int32),
        pltpu.VMEM((TAIL,), jnp.int32),
        pltpu.VMEM((CHUNK,), jnp.int32),
        pltpu.VMEM((NTAIL_N,), jnp.int32),
        pltpu.VMEM((CHUNK,), jnp.float32),
        pltpu.VMEM((80,), jnp.float32),
        pltpu.VMEM((M // NS,), jnp.float32),
        pltpu.VMEM_SHARED((N,), jnp.float32),
        pltpu.VMEM_SHARED((M,), jnp.float32),
        pltpu.VMEM_SHARED((M,), jnp.float32),
        pltpu.SemaphoreType.DMA,
    ]

    @functools.partial(pl.kernel, out_type=out_type, mesh=_mesh(),
                       scratch_types=scratch, name="sc_prep")
    def prep(srcx, dstx, etype, hier, ones_h, zn_h, zm_h,
             gbu_out, gmod_out, md_out, degb_out, degm_out, cnt_out,
             hs_v, hs_t, s_v, d_v, t_v, gbu_v, gmod_v, md_v,
             s_t, d_t, t_t, gbu_t, gmod_t, md_t, hidx_v, hidx_t,
             ones_v, stage_n, stage_m, degb_acc, degm_acc, cnt_acc, sem):
        c = lax.axis_index("c")
        s = lax.axis_index("s")
        w = c * NS + s
        rm = M // NS
        NZB = N // 80

        pltpu.sync_copy(zn_h, stage_n)
        for j in range((NZB + NS - 1) // NS):
            zid = s + NS * j

            @pl.when(zid < NZB)
            def _():
                pltpu.sync_copy(stage_n, degb_acc.at[pl.ds(zid * 80, 80)])
        pltpu.sync_copy(zm_h, stage_m)
        pltpu.sync_copy(stage_m, degm_acc.at[pl.ds(s * rm, rm)])
        pltpu.sync_copy(stage_m, cnt_acc.at[pl.ds(s * rm, rm)])
        plsc.subcore_barrier()

        pltpu.sync_copy(ones_h, ones_v)

        ebase = w * EW

        def echunk(j, carry):
            base = ebase + j * CHUNK
            pltpu.sync_copy(srcx.at[pl.ds(base, CHUNK)], s_v)
            pltpu.sync_copy(dstx.at[pl.ds(base, CHUNK)], d_v)
            pltpu.sync_copy(etype.at[pl.ds(base, CHUNK)], t_v)
            pltpu.async_copy(hier.at[s_v], hs_v, sem).wait()
            pltpu.async_copy(hier.at[d_v], md_v, sem).wait()

            def vec(i, carry2):
                sv = s_v[pl.ds(i * 16, 16)]
                tv = t_v[pl.ds(i * 16, 16)]
                gbu_v[pl.ds(i * 16, 16)] = tv * N + sv
                gmod_v[pl.ds(i * 16, 16)] = tv * M + hs_v[pl.ds(i * 16, 16)]
                return carry2

            lax.fori_loop(0, CHUNK // 16, vec, 0)
            pltpu.sync_copy(gbu_v, gbu_out.at[pl.ds(base, CHUNK)])
            pltpu.sync_copy(gmod_v, gmod_out.at[pl.ds(base, CHUNK)])
            pltpu.sync_copy(md_v, md_out.at[pl.ds(base, CHUNK)])
            pltpu.sync_copy(ones_v, degb_acc.at[d_v], add=True)
            pltpu.sync_copy(ones_v, degm_acc.at[md_v], add=True)
            return carry

        lax.fori_loop(0, NFULL, echunk, 0)

        tb = ebase + NFULL * CHUNK
        pltpu.sync_copy(srcx.at[pl.ds(tb, TAIL)], s_t)
        pltpu.sync_copy(dstx.at[pl.ds(tb, TAIL)], d_t)
        pltpu.sync_copy(etype.at[pl.ds(tb, TAIL)], t_t)
        pltpu.async_copy(hier.at[s_t], hs_t, sem).wait()
        pltpu.async_copy(hier.at[d_t], md_t, sem).wait()
        sv = s_t[...]
        tv = t_t[...]
        gbu_t[...] = tv * N + sv
        gmod_t[...] = tv * M + hs_t[...]
        pltpu.sync_copy(gbu_t, gbu_out.at[pl.ds(tb, TAIL)])
        pltpu.sync_copy(gmod_t, gmod_out.at[pl.ds(tb, TAIL)])
        pltpu.sync_copy(md_t, md_out.at[pl.ds(tb, TAIL)])
        pltpu.sync_copy(ones_v.at[pl.ds(0, TAIL)], degb_acc.at[d_t], add=True)
        pltpu.sync_copy(ones_v.at[pl.ds(0, TAIL)], degm_acc.at[md_t], add=True)

        for j in range((NCHUNK_N + NW - 1) // NW):
            cid = w + NW * j

            @pl.when(cid < NCHUNK_N)
            def _():
                pltpu.sync_copy(hier.at[pl.ds(cid * CHUNK, CHUNK)], hidx_v)
                pltpu.sync_copy(ones_v, cnt_acc.at[hidx_v], add=True)

        @pl.when(w == 0)
        def _():
            pltpu.sync_copy(hier.at[pl.ds(N - NTAIL_N, NTAIL_N)], hidx_t)
            pltpu.sync_copy(ones_v.at[pl.ds(0, NTAIL_N)], cnt_acc.at[hidx_t],
                            add=True)

        plsc.subcore_barrier()

        for j in range((NZB + NS - 1) // NS):
            zid = s + NS * j

            @pl.when(zid < NZB)
            def _():
                pltpu.sync_copy(degb_acc.at[pl.ds(zid * 80, 80)], stage_n)
                pltpu.sync_copy(stage_n,
                                degb_out.at[pl.ds(c * N + zid * 80, 80)])
        pltpu.sync_copy(degm_acc.at[pl.ds(s * rm, rm)], stage_m)
        pltpu.sync_copy(stage_m, degm_out.at[pl.ds(c * M + s * rm, rm)])
        pltpu.sync_copy(cnt_acc.at[pl.ds(s * rm, rm)], stage_m)
        pltpu.sync_copy(stage_m, cnt_out.at[pl.ds(c * M + s * rm, rm)])

    return prep


def _build_agg(NN, VT, ZCH, NZ, name):
    assert ZCH * NZ == NN
    scratch = [
        pltpu.VMEM_SHARED((NN, D), jnp.float32),
        pltpu.VMEM((CHUNK,), jnp.int32),
        pltpu.VMEM((CHUNK,), jnp.int32),
        pltpu.VMEM((CHUNK, D), jnp.float32),
        pltpu.VMEM((TAIL,), jnp.int32),
        pltpu.VMEM((TAIL,), jnp.int32),
        pltpu.VMEM((TAIL, D), jnp.float32),
        pltpu.VMEM((ZCH, D), jnp.float32),
        pltpu.SemaphoreType.DMA,
    ]

    @functools.partial(pl.kernel,
                       out_type=jax.ShapeDtypeStruct((NC * NN, D), jnp.float32),
                       mesh=_mesh(), scratch_types=scratch, name=name)
    def agg(table, gidx, dstx, zrows, out,
            acc, gi_v, di_v, rows_v, gi_t, di_t, rows_t, stage, sem):
        c = lax.axis_index("c")
        s = lax.axis_index("s")
        w = c * NS + s

        pltpu.sync_copy(zrows, stage)
        for j in range((NZ + NS - 1) // NS):
            cid = s + NS * j

            @pl.when(cid < NZ)
            def _():
                pltpu.sync_copy(stage, acc.at[pl.ds(cid * ZCH, ZCH)])
        plsc.subcore_barrier()

        ebase = w * EW

        def chunk(j, carry):
            base = ebase + j * CHUNK
            pltpu.sync_copy(gidx.at[pl.ds(base, CHUNK)], gi_v)
            pltpu.sync_copy(dstx.at[pl.ds(base, CHUNK)], di_v)
            pltpu.async_copy(table.at[gi_v], rows_v, sem).wait()
            pltpu.sync_copy(rows_v, acc.at[di_v], add=True)
            return carry

        lax.fori_loop(0, NFULL, chunk, 0)

        tb = ebase + NFULL * CHUNK
        pltpu.sync_copy(gidx.at[pl.ds(tb, TAIL)], gi_t)
        pltpu.sync_copy(dstx.at[pl.ds(tb, TAIL)], di_t)
        pltpu.async_copy(table.at[gi_t], rows_t, sem).wait()
        pltpu.sync_copy(rows_t, acc.at[di_t], add=True)

        plsc.subcore_barrier()
        for j in range((NZ + NS - 1) // NS):
            cid = s + NS * j

            @pl.when(cid < NZ)
            def _():
                pltpu.sync_copy(acc.at[pl.ds(cid * ZCH, ZCH)], stage)
                pltpu.sync_copy(stage, out.at[pl.ds(c * NN + cid * ZCH, ZCH)])

    return agg


def _build_pool():
    RC = 80
    NRC = N // RC
    ZCH = M // NS
    scratch = [
        pltpu.VMEM_SHARED((M, D), jnp.float32),
        pltpu.VMEM((RC, D), jnp.float32),
        pltpu.VMEM((RC,), jnp.int32),
        pltpu.VMEM((ZCH, D), jnp.float32),
    ]

    @functools.partial(pl.kernel,
                       out_type=jax.ShapeDtypeStruct((NC * M, D), jnp.float32),
                       mesh=_mesh(), scratch_types=scratch, name="sc_pool")
    def pool(h, hier, zrows, out, acc, rows_v, hidx_v, stage):
        c = lax.axis_index("c")
        s = lax.axis_index("s")
        w = c * NS + s

        pltpu.sync_copy(zrows, stage)
        pltpu.sync_copy(stage, acc.at[pl.ds(s * ZCH, ZCH)])
        plsc.subcore_barrier()

        for j in range((NRC + NW - 1) // NW):
            cid = w + NW * j

            @pl.when(cid < NRC)
            def _():
                base = cid * RC
                pltpu.sync_copy(h.at[pl.ds(base, RC)], rows_v)
                pltpu.sync_copy(hier.at[pl.ds(base, RC)], hidx_v)
                pltpu.sync_copy(rows_v, acc.at[hidx_v], add=True)

        plsc.subcore_barrier()
        pltpu.sync_copy(acc.at[pl.ds(s * ZCH, ZCH)], stage)
        pltpu.sync_copy(stage, out.at[pl.ds(c * M + s * ZCH, ZCH)])

    return pool


def _dot(a, b):
    return jnp.dot(a, b, preferred_element_type=jnp.float32)


def _head_body(x_ref, wi_ref, bi_ref, wrel_ref, wself_ref, b_ref,
               hr_ref, sb_ref):
    h = jnp.maximum(_dot(x_ref[...], wi_ref[...]) + bi_ref[...], 0.0)
    for r in range(R):
        hr_ref[r] = _dot(h, wrel_ref[r])
    sb_ref[...] = _dot(h, wself_ref[...]) + b_ref[...]


def _norm_h(p_ref, deg_ref, sbp_ref):
    degs = deg_ref[0, :, 0:1] + deg_ref[1, :, 0:1]
    inv = 1.0 / jnp.maximum(degs, 1.0)
    return jnp.maximum((p_ref[0] + p_ref[1]) * inv + sbp_ref[...], 0.0)


def _mid_body(p_ref, deg_ref, sbp_ref, wrel_ref, wself_ref, b_ref,
              hr_ref, sb_ref):
    h = _norm_h(p_ref, deg_ref, sbp_ref)
    for r in range(R):
        hr_ref[r] = _dot(h, wrel_ref[r])
    sb_ref[...] = _dot(h, wself_ref[...]) + b_ref[...]


def _combine_body(p_ref, deg_ref, sbp_ref, h_ref):
    h_ref[...] = _norm_h(p_ref, deg_ref, sbp_ref)


def _mod0_body(p_ref, cnt_ref, wrel_ref, wself_ref, b_ref, hr_ref, sb_ref):
    cnts = cnt_ref[0, :, 0:1] + cnt_ref[1, :, 0:1]
    pooled = (p_ref[0] + p_ref[1]) * (1.0 / jnp.maximum(cnts, 1.0))
    for r in range(R):
        hr_ref[r] = _dot(pooled, wrel_ref[r])
    sb_ref[...] = _dot(pooled, wself_ref[...]) + b_ref[...]


def _tail_body(p_ref, deg_ref, sbp_ref, wf_ref, bf_ref, out_ref):
    h = _norm_h(p_ref, deg_ref, sbp_ref)
    out_ref[...] = jnp.maximum(_dot(h, wf_ref[...]) + bf_ref[...], 0.0)


_BN = 1000


def _head_call(x, wi, bi, wrel, wself, b):
    nb = N // _BN
    return pl.pallas_call(
        _head_body,
        grid=(nb,),
        in_specs=[
            pl.BlockSpec((_BN, D), lambda i: (i, 0)),
            pl.BlockSpec((D, D), lambda i: (0, 0)),
            pl.BlockSpec((1, D), lambda i: (0, 0)),
            pl.BlockSpec((R, D, D), lambda i: (0, 0, 0)),
            pl.BlockSpec((D, D), lambda i: (0, 0)),
            pl.BlockSpec((1, D), lambda i: (0, 0)),
        ],
        out_specs=[
            pl.BlockSpec((R, _BN, D), lambda i: (0, i, 0)),
            pl.BlockSpec((_BN, D), lambda i: (i, 0)),
        ],
        out_shape=[
            jax.ShapeDtypeStruct((R, N, D), jnp.float32),
            jax.ShapeDtypeStruct((N, D), jnp.float32),
        ],
    )(x, wi, bi.reshape(1, D), wrel, wself, b.reshape(1, D))


def _mid_call(part, deg2, sbp, wrel, wself, b):
    nb = N // _BN
    return pl.pallas_call(
        _mid_body,
        grid=(nb,),
        in_specs=[
            pl.BlockSpec((NC, _BN, D), lambda i: (0, i, 0)),
            pl.BlockSpec((NC, _BN, 1), lambda i: (0, i, 0)),
            pl.BlockSpec((_BN, D), lambda i: (i, 0)),
            pl.BlockSpec((R, D, D), lambda i: (0, 0, 0)),
            pl.BlockSpec((D, D), lambda i: (0, 0)),
            pl.BlockSpec((1, D), lambda i: (0, 0)),
        ],
        out_specs=[
            pl.BlockSpec((R, _BN, D), lambda i: (0, i, 0)),
            pl.BlockSpec((_BN, D), lambda i: (i, 0)),
        ],
        out_shape=[
            jax.ShapeDtypeStruct((R, N, D), jnp.float32),
            jax.ShapeDtypeStruct((N, D), jnp.float32),
        ],
    )(part.reshape(NC, N, D), deg2.reshape(NC, N, 1), sbp,
      wrel, wself, b.reshape(1, D))


def _combine_call(part, deg2, sbp):
    nb = N // _BN
    return pl.pallas_call(
        _combine_body,
        grid=(nb,),
        in_specs=[
            pl.BlockSpec((NC, _BN, D), lambda i: (0, i, 0)),
            pl.BlockSpec((NC, _BN, 1), lambda i: (0, i, 0)),
            pl.BlockSpec((_BN, D), lambda i: (i, 0)),
        ],
        out_specs=pl.BlockSpec((_BN, D), lambda i: (i, 0)),
        out_shape=jax.ShapeDtypeStruct((N, D), jnp.float32),
    )(part.reshape(NC, N, D), deg2.reshape(NC, N, 1), sbp)


def _mod0_call(pool2, cnt2, wrel, wself, b):
    return pl.pallas_call(
        _mod0_body,
        out_shape=[
            jax.ShapeDtypeStruct((R, M, D), jnp.float32),
            jax.ShapeDtypeStruct((M, D), jnp.float32),
        ],
    )(pool2.reshape(NC, M, D), cnt2.reshape(NC, M, 1),
      wrel, wself, b.reshape(1, D))


def _modmid_call(part, deg2, sbp, wrel, wself, b):
    return pl.pallas_call(
        _mid_body,
        out_shape=[
            jax.ShapeDtypeStruct((R, M, D), jnp.float32),
            jax.ShapeDtypeStruct((M, D), jnp.float32),
        ],
    )(part.reshape(NC, M, D), deg2.reshape(NC, M, 1), sbp,
      wrel, wself, b.reshape(1, D))


def _tail_call(part, deg2, sbp, wf, bf):
    return pl.pallas_call(
        _tail_body,
        out_shape=jax.ShapeDtypeStruct((M, D), jnp.float32),
    )(part.reshape(NC, M, D), deg2.reshape(NC, M, 1), sbp,
      wf, bf.reshape(1, D))


_prep = _build_prep()
_agg_n = _build_agg(N, R * N, 80, 125, "sc_agg_n")
_agg_m = _build_agg(M, R * M, 64, 16, "sc_agg_m")
_pool = _build_pool()


def kernel(x, edge_index, edge_type, hierarchy,
           W_init, b_init, W_rel_bu, W_self_bu, b_bu,
           W_rel_mod, W_self_mod, b_mod, W_fin, b_fin):
    srcx = edge_index[0]
    dstx = edge_index[1]
    ones_h = jnp.ones((CHUNK,), jnp.float32)
    zn_h = jnp.zeros((80,), jnp.float32)
    zm_h = jnp.zeros((M // NS,), jnp.float32)
    z80d = jnp.zeros((80, D), jnp.float32)
    z64d = jnp.zeros((64, D), jnp.float32)

    gbu, gmod, mdst, degb2, degm2, cnt2 = _prep(srcx, dstx, edge_type,
                                                hierarchy, ones_h, zn_h, zm_h)

    hr, sb = _head_call(x, W_init, b_init, W_rel_bu[0], W_self_bu[0], b_bu[0])
    part = _agg_n(hr.reshape(R * N, D), gbu, dstx, z80d)
    hr, sb = _mid_call(part, degb2, sb, W_rel_bu[1], W_self_bu[1], b_bu[1])
    part = _agg_n(hr.reshape(R * N, D), gbu, dstx, z80d)
    h2 = _combine_call(part, degb2, sb)

    pool2 = _pool(h2, hierarchy, z64d)
    hrm, sbm = _mod0_call(pool2, cnt2, W_rel_mod[0], W_self_mod[0], b_mod[0])
    mp = _agg_m(hrm.reshape(R * M, D), gmod, mdst, z64d)
    hrm, sbm = _modmid_call(mp, degm2, sbm, W_rel_mod[1], W_self_mod[1],
                            b_mod[1])
    mp = _agg_m(hrm.reshape(R * M, D), gmod, mdst, z64d)

    return _tail_call(mp, degm2, sbm, W_fin, b_fin)


# speedup vs baseline: 33.1757x; 1.5144x over previous
import functools

import jax
import jax.numpy as jnp
from jax import lax
from jax.experimental import pallas as pl
from jax.experimental.pallas import tpu as pltpu
from jax.experimental.pallas import tpu_sc as plsc

N = 10000
E = 320000
D = 128
R = 4
M = 1024

NC = 2
NS = 16
NW = NC * NS
EW = E // NW
CHUNK = 128
NFULL = EW // CHUNK
TAIL = EW - NFULL * CHUNK
CW = 16
NCHUNK_N = N // CHUNK
NTAIL_N = N - NCHUNK_N * CHUNK


def _mesh():
    return plsc.VectorSubcoreMesh(core_axis_name="c", subcore_axis_name="s")


def _build_prep():
    out_type = (
        jax.ShapeDtypeStruct((E,), jnp.int32),
        jax.ShapeDtypeStruct((E,), jnp.int32),
        jax.ShapeDtypeStruct((E,), jnp.int32),
        jax.ShapeDtypeStruct((NC * N,), jnp.float32),
        jax.ShapeDtypeStruct((NC * M,), jnp.float32),
        jax.ShapeDtypeStruct((NC * M,), jnp.float32),
    )
    scratch = [
        pltpu.VMEM((CHUNK,), jnp.int32),
        pltpu.VMEM((TAIL,), jnp.int32),
        pltpu.VMEM((CHUNK,), jnp.int32),
        pltpu.VMEM((CHUNK,), jnp.int32),
        pltpu.VMEM((CHUNK,), jnp.int32),
        pltpu.VMEM((CHUNK,), jnp.int32),
        pltpu.VMEM((CHUNK,), jnp.int32),
        pltpu.VMEM((CHUNK,), jnp.int32),
        pltpu.VMEM((TAIL,), jnp.int32),
        pltpu.VMEM((TAIL,), jnp.int32),
        pltpu.VMEM((TAIL,), jnp.int32),
        pltpu.VMEM((TAIL,), jnp.int32),
        pltpu.VMEM((TAIL,), jnp.int32),
        pltpu.VMEM((TAIL,), jnp.int32),
        pltpu.VMEM((CHUNK,), jnp.int32),
        pltpu.VMEM((NTAIL_N,), jnp.int32),
        pltpu.VMEM((CHUNK,), jnp.float32),
        pltpu.VMEM((80,), jnp.float32),
        pltpu.VMEM((M // NS,), jnp.float32),
        pltpu.VMEM_SHARED((N,), jnp.float32),
        pltpu.VMEM_SHARED((M,), jnp.float32),
        pltpu.VMEM_SHARED((M,), jnp.float32),
        pltpu.SemaphoreType.DMA,
    ]

    @functools.partial(pl.kernel, out_type=out_type, mesh=_mesh(),
                       scratch_types=scratch, name="sc_prep")
    def prep(srcx, dstx, etype, hier, ones_h, zn_h, zm_h,
             gbu_out, gmod_out, md_out, degb_out, degm_out, cnt_out,
             hs_v, hs_t, s_v, d_v, t_v, gbu_v, gmod_v, md_v,
             s_t, d_t, t_t, gbu_t, gmod_t, md_t, hidx_v, hidx_t,
             ones_v, stage_n, stage_m, degb_acc, degm_acc, cnt_acc, sem):
        c = lax.axis_index("c")
        s = lax.axis_index("s")
        w = c * NS + s
        rm = M // NS
        NZB = N // 80

        pltpu.sync_copy(zn_h, stage_n)
        for j in range((NZB + NS - 1) // NS):
            zid = s + NS * j

            @pl.when(zid < NZB)
            def _():
                pltpu.sync_copy(stage_n, degb_acc.at[pl.ds(zid * 80, 80)])
        pltpu.sync_copy(zm_h, stage_m)
        pltpu.sync_copy(stage_m, degm_acc.at[pl.ds(s * rm, rm)])
        pltpu.sync_copy(stage_m, cnt_acc.at[pl.ds(s * rm, rm)])
        plsc.subcore_barrier()

        pltpu.sync_copy(ones_h, ones_v)

        ebase = w * EW

        def echunk(j, carry):
            base = ebase + j * CHUNK
            pltpu.sync_copy(srcx.at[pl.ds(base, CHUNK)], s_v)
            pltpu.sync_copy(dstx.at[pl.ds(base, CHUNK)], d_v)
            pltpu.sync_copy(etype.at[pl.ds(base, CHUNK)], t_v)
            pltpu.async_copy(hier.at[s_v], hs_v, sem).wait()
            pltpu.async_copy(hier.at[d_v], md_v, sem).wait()

            def vec(i, carry2):
                sv = s_v[pl.ds(i * 16, 16)]
                tv = t_v[pl.ds(i * 16, 16)]
                gbu_v[pl.ds(i * 16, 16)] = tv * N + sv
                gmod_v[pl.ds(i * 16, 16)] = tv * M + hs_v[pl.ds(i * 16, 16)]
                return carry2

            lax.fori_loop(0, CHUNK // 16, vec, 0)
            pltpu.sync_copy(gbu_v, gbu_out.at[pl.ds(base, CHUNK)])
            pltpu.sync_copy(gmod_v, gmod_out.at[pl.ds(base, CHUNK)])
            pltpu.sync_copy(md_v, md_out.at[pl.ds(base, CHUNK)])
            pltpu.sync_copy(ones_v, degb_acc.at[d_v], add=True)
            pltpu.sync_copy(ones_v, degm_acc.at[md_v], add=True)
            return carry

        lax.fori_loop(0, NFULL, echunk, 0)

        tb = ebase + NFULL * CHUNK
        pltpu.sync_copy(srcx.at[pl.ds(tb, TAIL)], s_t)
        pltpu.sync_copy(dstx.at[pl.ds(tb, TAIL)], d_t)
        pltpu.sync_copy(etype.at[pl.ds(tb, TAIL)], t_t)
        pltpu.async_copy(hier.at[s_t], hs_t, sem).wait()
        pltpu.async_copy(hier.at[d_t], md_t, sem).wait()
        sv = s_t[...]
        tv = t_t[...]
        gbu_t[...] = tv * N + sv
        gmod_t[...] = tv * M + hs_t[...]
        pltpu.sync_copy(gbu_t, gbu_out.at[pl.ds(tb, TAIL)])
        pltpu.sync_copy(gmod_t, gmod_out.at[pl.ds(tb, TAIL)])
        pltpu.sync_copy(md_t, md_out.at[pl.ds(tb, TAIL)])
        pltpu.sync_copy(ones_v.at[pl.ds(0, TAIL)], degb_acc.at[d_t], add=True)
        pltpu.sync_copy(ones_v.at[pl.ds(0, TAIL)], degm_acc.at[md_t], add=True)

        for j in range((NCHUNK_N + NW - 1) // NW):
            cid = w + NW * j

            @pl.when(cid < NCHUNK_N)
            def _():
                pltpu.sync_copy(hier.at[pl.ds(cid * CHUNK, CHUNK)], hidx_v)
                pltpu.sync_copy(ones_v, cnt_acc.at[hidx_v], add=True)

        @pl.when(w == 0)
        def _():
            pltpu.sync_copy(hier.at[pl.ds(N - NTAIL_N, NTAIL_N)], hidx_t)
            pltpu.sync_copy(ones_v.at[pl.ds(0, NTAIL_N)], cnt_acc.at[hidx_t],
                            add=True)

        plsc.subcore_barrier()

        for j in range((NZB + NS - 1) // NS):
            zid = s + NS * j

            @pl.when(zid < NZB)
            def _():
                pltpu.sync_copy(degb_acc.at[pl.ds(zid * 80, 80)], stage_n)
                pltpu.sync_copy(stage_n,
                                degb_out.at[pl.ds(c * N + zid * 80, 80)])
        pltpu.sync_copy(degm_acc.at[pl.ds(s * rm, rm)], stage_m)
        pltpu.sync_copy(stage_m, degm_out.at[pl.ds(c * M + s * rm, rm)])
        pltpu.sync_copy(cnt_acc.at[pl.ds(s * rm, rm)], stage_m)
        pltpu.sync_copy(stage_m, cnt_out.at[pl.ds(c * M + s * rm, rm)])

    return prep


def _build_agg(NN, VT, ZCH, NZ, name):
    assert ZCH * NZ == NN
    scratch = [
        pltpu.VMEM_SHARED((NN, D), jnp.float32),
        pltpu.VMEM((CHUNK,), jnp.int32),
        pltpu.VMEM((CHUNK,), jnp.int32),
        pltpu.VMEM((CHUNK,), jnp.int32),
        pltpu.VMEM((CHUNK,), jnp.int32),
        pltpu.VMEM((CHUNK, D), jnp.float32),
        pltpu.VMEM((CHUNK, D), jnp.float32),
        pltpu.VMEM((TAIL,), jnp.int32),
        pltpu.VMEM((TAIL,), jnp.int32),
        pltpu.VMEM((TAIL, D), jnp.float32),
        pltpu.VMEM((ZCH, D), jnp.float32),
        pltpu.SemaphoreType.DMA,
        pltpu.SemaphoreType.DMA,
        pltpu.SemaphoreType.DMA,
        pltpu.SemaphoreType.DMA,
        pltpu.SemaphoreType.DMA,
        pltpu.SemaphoreType.DMA,
    ]

    @functools.partial(pl.kernel,
                       out_type=jax.ShapeDtypeStruct((NC * NN, D), jnp.float32),
                       mesh=_mesh(), scratch_types=scratch, name=name)
    def agg(table, gidx, dstx, zrows, out,
            acc, gi0, gi1, di0, di1, rows0, rows1,
            gi_t, di_t, rows_t, stage, sgi0, sgi1, sdi0, sdi1, sr0, sr1):
        c = lax.axis_index("c")
        s = lax.axis_index("s")
        w = c * NS + s
        gis = (gi0, gi1)
        dis = (di0, di1)
        rows = (rows0, rows1)
        sgis = (sgi0, sgi1)
        sdis = (sdi0, sdi1)
        srs = (sr0, sr1)

        pltpu.sync_copy(zrows, stage)
        for j in range((NZ + NS - 1) // NS):
            cid = s + NS * j

            @pl.when(cid < NZ)
            def _():
                pltpu.sync_copy(stage, acc.at[pl.ds(cid * ZCH, ZCH)])
        plsc.subcore_barrier()

        ebase = w * EW

        def issue_idx(j, b):
            base = ebase + j * CHUNK
            pltpu.async_copy(gidx.at[pl.ds(base, CHUNK)], gis[b], sgis[b])
            pltpu.async_copy(dstx.at[pl.ds(base, CHUNK)], dis[b], sdis[b])

        def wait_idx(j, b):
            base = ebase + j * CHUNK
            pltpu.make_async_copy(gidx.at[pl.ds(base, CHUNK)], gis[b],
                                  sgis[b]).wait()
            pltpu.make_async_copy(dstx.at[pl.ds(base, CHUNK)], dis[b],
                                  sdis[b]).wait()

        def issue_gather(b):
            pltpu.async_copy(table.at[gis[b]], rows[b], srs[b])

        def wait_gather(b):
            pltpu.make_async_copy(table.at[gis[b]], rows[b], srs[b]).wait()

        issue_idx(0, 0)
        issue_idx(1, 1)
        wait_idx(0, 0)
        issue_gather(0)

        def body(jj, carry):
            j = 2 * jj
            for ph in range(2):
                b = ph
                nb = 1 - ph
                wait_gather(b)

                @pl.when(j + ph + 1 < NFULL)
                def _():
                    wait_idx(j + ph + 1, nb)
                    issue_gather(nb)

                pltpu.sync_copy(rows[b], acc.at[dis[b]], add=True)

                @pl.when(j + ph + 2 < NFULL)
                def _():
                    issue_idx(j + ph + 2, b)
            return carry

        lax.fori_loop(0, NFULL // 2, body, 0)

        tb = ebase + NFULL * CHUNK
        pltpu.sync_copy(gidx.at[pl.ds(tb, TAIL)], gi_t)
        pltpu.sync_copy(dstx.at[pl.ds(tb, TAIL)], di_t)
        pltpu.async_copy(table.at[gi_t], rows_t, sr0).wait()
        pltpu.sync_copy(rows_t, acc.at[di_t], add=True)

        plsc.subcore_barrier()
        for j in range((NZ + NS - 1) // NS):
            cid = s + NS * j

            @pl.when(cid < NZ)
            def _():
                pltpu.sync_copy(acc.at[pl.ds(cid * ZCH, ZCH)], stage)
                pltpu.sync_copy(stage, out.at[pl.ds(c * NN + cid * ZCH, ZCH)])

    return agg


def _build_pool():
    RC = 80
    NRC = N // RC
    ZCH = M // NS
    scratch = [
        pltpu.VMEM_SHARED((M, D), jnp.float32),
        pltpu.VMEM((RC, D), jnp.float32),
        pltpu.VMEM((RC,), jnp.int32),
        pltpu.VMEM((ZCH, D), jnp.float32),
    ]

    @functools.partial(pl.kernel,
                       out_type=jax.ShapeDtypeStruct((NC * M, D), jnp.float32),
                       mesh=_mesh(), scratch_types=scratch, name="sc_pool")
    def pool(h, hier, zrows, out, acc, rows_v, hidx_v, stage):
        c = lax.axis_index("c")
        s = lax.axis_index("s")
        w = c * NS + s

        pltpu.sync_copy(zrows, stage)
        pltpu.sync_copy(stage, acc.at[pl.ds(s * ZCH, ZCH)])
        plsc.subcore_barrier()

        for j in range((NRC + NW - 1) // NW):
            cid = w + NW * j

            @pl.when(cid < NRC)
            def _():
                base = cid * RC
                pltpu.sync_copy(h.at[pl.ds(base, RC)], rows_v)
                pltpu.sync_copy(hier.at[pl.ds(base, RC)], hidx_v)
                pltpu.sync_copy(rows_v, acc.at[hidx_v], add=True)

        plsc.subcore_barrier()
        pltpu.sync_copy(acc.at[pl.ds(s * ZCH, ZCH)], stage)
        pltpu.sync_copy(stage, out.at[pl.ds(c * M + s * ZCH, ZCH)])

    return pool


def _dot(a, b):
    return jnp.dot(a, b, preferred_element_type=jnp.float32)


def _head_body(x_ref, wi_ref, bi_ref, wrel_ref, wself_ref, b_ref,
               hr_ref, sb_ref):
    h = jnp.maximum(_dot(x_ref[...], wi_ref[...]) + bi_ref[...], 0.0)
    for r in range(R):
        hr_ref[r] = _dot(h, wrel_ref[r])
    sb_ref[...] = _dot(h, wself_ref[...]) + b_ref[...]


def _norm_h(p_ref, deg_ref, sbp_ref):
    degs = deg_ref[0, :, 0:1] + deg_ref[1, :, 0:1]
    inv = 1.0 / jnp.maximum(degs, 1.0)
    return jnp.maximum((p_ref[0] + p_ref[1]) * inv + sbp_ref[...], 0.0)


def _mid_body(p_ref, deg_ref, sbp_ref, wrel_ref, wself_ref, b_ref,
              hr_ref, sb_ref):
    h = _norm_h(p_ref, deg_ref, sbp_ref)
    for r in range(R):
        hr_ref[r] = _dot(h, wrel_ref[r])
    sb_ref[...] = _dot(h, wself_ref[...]) + b_ref[...]


def _combine_body(p_ref, deg_ref, sbp_ref, h_ref):
    h_ref[...] = _norm_h(p_ref, deg_ref, sbp_ref)


def _mod0_body(p_ref, cnt_ref, wrel_ref, wself_ref, b_ref, hr_ref, sb_ref):
    cnts = cnt_ref[0, :, 0:1] + cnt_ref[1, :, 0:1]
    pooled = (p_ref[0] + p_ref[1]) * (1.0 / jnp.maximum(cnts, 1.0))
    for r in range(R):
        hr_ref[r] = _dot(pooled, wrel_ref[r])
    sb_ref[...] = _dot(pooled, wself_ref[...]) + b_ref[...]


def _tail_body(p_ref, deg_ref, sbp_ref, wf_ref, bf_ref, out_ref):
    h = _norm_h(p_ref, deg_ref, sbp_ref)
    out_ref[...] = jnp.maximum(_dot(h, wf_ref[...]) + bf_ref[...], 0.0)


_BN = 1000


def _head_call(x, wi, bi, wrel, wself, b):
    nb = N // _BN
    return pl.pallas_call(
        _head_body,
        grid=(nb,),
        in_specs=[
            pl.BlockSpec((_BN, D), lambda i: (i, 0)),
            pl.BlockSpec((D, D), lambda i: (0, 0)),
            pl.BlockSpec((1, D), lambda i: (0, 0)),
            pl.BlockSpec((R, D, D), lambda i: (0, 0, 0)),
            pl.BlockSpec((D, D), lambda i: (0, 0)),
            pl.BlockSpec((1, D), lambda i: (0, 0)),
        ],
        out_specs=[
            pl.BlockSpec((R, _BN, D), lambda i: (0, i, 0)),
            pl.BlockSpec((_BN, D), lambda i: (i, 0)),
        ],
        out_shape=[
            jax.ShapeDtypeStruct((R, N, D), jnp.float32),
            jax.ShapeDtypeStruct((N, D), jnp.float32),
        ],
    )(x, wi, bi.reshape(1, D), wrel, wself, b.reshape(1, D))


def _mid_call(part, deg2, sbp, wrel, wself, b):
    nb = N // _BN
    return pl.pallas_call(
        _mid_body,
        grid=(nb,),
        in_specs=[
            pl.BlockSpec((NC, _BN, D), lambda i: (0, i, 0)),
            pl.BlockSpec((NC, _BN, 1), lambda i: (0, i, 0)),
            pl.BlockSpec((_BN, D), lambda i: (i, 0)),
            pl.BlockSpec((R, D, D), lambda i: (0, 0, 0)),
            pl.BlockSpec((D, D), lambda i: (0, 0)),
            pl.BlockSpec((1, D), lambda i: (0, 0)),
        ],
        out_specs=[
            pl.BlockSpec((R, _BN, D), lambda i: (0, i, 0)),
            pl.BlockSpec((_BN, D), lambda i: (i, 0)),
        ],
        out_shape=[
            jax.ShapeDtypeStruct((R, N, D), jnp.float32),
            jax.ShapeDtypeStruct((N, D), jnp.float32),
        ],
    )(part.reshape(NC, N, D), deg2.reshape(NC, N, 1), sbp,
      wrel, wself, b.reshape(1, D))


def _combine_call(part, deg2, sbp):
    nb = N // _BN
    return pl.pallas_call(
        _combine_body,
        grid=(nb,),
        in_specs=[
            pl.BlockSpec((NC, _BN, D), lambda i: (0, i, 0)),
            pl.BlockSpec((NC, _BN, 1), lambda i: (0, i, 0)),
            pl.BlockSpec((_BN, D), lambda i: (i, 0)),
        ],
        out_specs=pl.BlockSpec((_BN, D), lambda i: (i, 0)),
        out_shape=jax.ShapeDtypeStruct((N, D), jnp.float32),
    )(part.reshape(NC, N, D), deg2.reshape(NC, N, 1), sbp)


def _mod0_call(pool2, cnt2, wrel, wself, b):
    return pl.pallas_call(
        _mod0_body,
        out_shape=[
            jax.ShapeDtypeStruct((R, M, D), jnp.float32),
            jax.ShapeDtypeStruct((M, D), jnp.float32),
        ],
    )(pool2.reshape(NC, M, D), cnt2.reshape(NC, M, 1),
      wrel, wself, b.reshape(1, D))


def _modmid_call(part, deg2, sbp, wrel, wself, b):
    return pl.pallas_call(
        _mid_body,
        out_shape=[
            jax.ShapeDtypeStruct((R, M, D), jnp.float32),
            jax.ShapeDtypeStruct((M, D), jnp.float32),
        ],
    )(part.reshape(NC, M, D), deg2.reshape(NC, M, 1), sbp,
      wrel, wself, b.reshape(1, D))


def _tail_call(part, deg2, sbp, wf, bf):
    return pl.pallas_call(
        _tail_body,
        out_shape=jax.ShapeDtypeStruct((M, D), jnp.float32),
    )(part.reshape(NC, M, D), deg2.reshape(NC, M, 1), sbp,
      wf, bf.reshape(1, D))


_prep = _build_prep()
_agg_n = _build_agg(N, R * N, 80, 125, "sc_agg_n")
_agg_m = _build_agg(M, R * M, 64, 16, "sc_agg_m")
_pool = _build_pool()


def kernel(x, edge_index, edge_type, hierarchy,
           W_init, b_init, W_rel_bu, W_self_bu, b_bu,
           W_rel_mod, W_self_mod, b_mod, W_fin, b_fin):
    srcx = edge_index[0]
    dstx = edge_index[1]
    ones_h = jnp.ones((CHUNK,), jnp.float32)
    zn_h = jnp.zeros((80,), jnp.float32)
    zm_h = jnp.zeros((M // NS,), jnp.float32)
    z80d = jnp.zeros((80, D), jnp.float32)
    z64d = jnp.zeros((64, D), jnp.float32)

    gbu, gmod, mdst, degb2, degm2, cnt2 = _prep(srcx, dstx, edge_type,
                                                hierarchy, ones_h, zn_h, zm_h)

    hr, sb = _head_call(x, W_init, b_init, W_rel_bu[0], W_self_bu[0], b_bu[0])
    part = _agg_n(hr.reshape(R * N, D), gbu, dstx, z80d)
    hr, sb = _mid_call(part, degb2, sb, W_rel_bu[1], W_self_bu[1], b_bu[1])
    part = _agg_n(hr.reshape(R * N, D), gbu, dstx, z80d)
    h2 = _combine_call(part, degb2, sb)

    pool2 = _pool(h2, hierarchy, z64d)
    hrm, sbm = _mod0_call(pool2, cnt2, W_rel_mod[0], W_self_mod[0], b_mod[0])
    mp = _agg_m(hrm.reshape(R * M, D), gmod, mdst, z64d)
    hrm, sbm = _modmid_call(mp, degm2, sbm, W_rel_mod[1], W_self_mod[1],
                            b_mod[1])
    mp = _agg_m(hrm.reshape(R * M, D), gmod, mdst, z64d)

    return _tail_call(mp, degm2, sbm, W_fin, b_fin)


# speedup vs baseline: 40.6978x; 1.2267x over previous
import functools

import jax
import jax.numpy as jnp
from jax import lax
from jax.experimental import pallas as pl
from jax.experimental.pallas import tpu as pltpu
from jax.experimental.pallas import tpu_sc as plsc

N = 10000
E = 320000
D = 128
R = 4
M = 1024

NC = 2
NS = 16
NW = NC * NS
EW = E // NW
CHUNK = 128
NFULL = EW // CHUNK
TAIL = EW - NFULL * CHUNK
CW = 16
NCHUNK_N = N // CHUNK
NTAIL_N = N - NCHUNK_N * CHUNK


def _mesh():
    return plsc.VectorSubcoreMesh(core_axis_name="c", subcore_axis_name="s")


def _build_prep():
    out_type = (
        jax.ShapeDtypeStruct((E,), jnp.int32),
        jax.ShapeDtypeStruct((E,), jnp.int32),
        jax.ShapeDtypeStruct((E,), jnp.int32),
        jax.ShapeDtypeStruct((NC * N,), jnp.float32),
        jax.ShapeDtypeStruct((NC * M,), jnp.float32),
        jax.ShapeDtypeStruct((NC * M,), jnp.float32),
    )
    scratch = [
        pltpu.VMEM((CHUNK,), jnp.int32),
        pltpu.VMEM((CHUNK,), jnp.int32),
        pltpu.VMEM((TAIL,), jnp.int32),
        pltpu.VMEM((CHUNK,), jnp.int32),
        pltpu.VMEM((CHUNK,), jnp.int32),
        pltpu.VMEM((CHUNK,), jnp.int32),
        pltpu.VMEM((CHUNK,), jnp.int32),
        pltpu.VMEM((CHUNK,), jnp.int32),
        pltpu.VMEM((CHUNK,), jnp.int32),
        pltpu.VMEM((CHUNK,), jnp.int32),
        pltpu.VMEM((CHUNK,), jnp.int32),
        pltpu.VMEM((CHUNK,), jnp.int32),
        pltpu.VMEM((CHUNK,), jnp.int32),
        pltpu.VMEM((CHUNK,), jnp.int32),
        pltpu.VMEM((CHUNK,), jnp.int32),
        pltpu.VMEM((TAIL,), jnp.int32),
        pltpu.VMEM((TAIL,), jnp.int32),
        pltpu.VMEM((TAIL,), jnp.int32),
        pltpu.VMEM((TAIL,), jnp.int32),
        pltpu.VMEM((TAIL,), jnp.int32),
        pltpu.VMEM((TAIL,), jnp.int32),
        pltpu.VMEM((CHUNK,), jnp.int32),
        pltpu.VMEM((NTAIL_N,), jnp.int32),
        pltpu.VMEM((CHUNK,), jnp.float32),
        pltpu.VMEM((80,), jnp.float32),
        pltpu.VMEM((M // NS,), jnp.float32),
        pltpu.VMEM_SHARED((N,), jnp.float32),
        pltpu.VMEM_SHARED((M,), jnp.float32),
        pltpu.VMEM_SHARED((M,), jnp.float32),
        pltpu.SemaphoreType.DMA,
        pltpu.SemaphoreType.DMA,
        pltpu.SemaphoreType.DMA,
        pltpu.SemaphoreType.DMA,
        pltpu.SemaphoreType.DMA,
        pltpu.SemaphoreType.DMA,
        pltpu.SemaphoreType.DMA,
    ]

    @functools.partial(pl.kernel, out_type=out_type, mesh=_mesh(),
                       scratch_types=scratch, name="sc_prep")
    def prep(srcx, dstx, etype, hier, ones_h, zn_h, zm_h,
             gbu_out, gmod_out, md_out, degb_out, degm_out, cnt_out,
             hs0, hs1, hs_t, s0, s1, d0, d1, t0, t1,
             gbu0, gbu1, gmod0, gmod1, md0, md1,
             s_t, d_t, t_t, gbu_t, gmod_t, md_t, hidx_v, hidx_t,
             ones_v, stage_n, stage_m, degb_acc, degm_acc, cnt_acc,
             sem, ssd0, ssd1, sg0, sg1, so0, so1):
        c = lax.axis_index("c")
        s = lax.axis_index("s")
        w = c * NS + s
        rm = M // NS
        NZB = N // 80

        pltpu.sync_copy(zn_h, stage_n)
        for j in range((NZB + NS - 1) // NS):
            zid = s + NS * j

            @pl.when(zid < NZB)
            def _():
                pltpu.sync_copy(stage_n, degb_acc.at[pl.ds(zid * 80, 80)])
        pltpu.sync_copy(zm_h, stage_m)
        pltpu.sync_copy(stage_m, degm_acc.at[pl.ds(s * rm, rm)])
        pltpu.sync_copy(stage_m, cnt_acc.at[pl.ds(s * rm, rm)])
        plsc.subcore_barrier()

        pltpu.sync_copy(ones_h, ones_v)

        ebase = w * EW
        ss = (s0, s1)
        dd = (d0, d1)
        tt = (t0, t1)
        hh = (hs0, hs1)
        mm = (md0, md1)
        gb = (gbu0, gbu1)
        gm = (gmod0, gmod1)
        ssd = (ssd0, ssd1)
        sg = (sg0, sg1)
        so = (so0, so1)

        def issue_sdt(j, b):
            base = ebase + j * CHUNK
            pltpu.async_copy(srcx.at[pl.ds(base, CHUNK)], ss[b], ssd[b])
            pltpu.async_copy(dstx.at[pl.ds(base, CHUNK)], dd[b], ssd[b])
            pltpu.async_copy(etype.at[pl.ds(base, CHUNK)], tt[b], ssd[b])

        def wait_sdt(j, b):
            base = ebase + j * CHUNK
            pltpu.make_async_copy(srcx.at[pl.ds(base, CHUNK)], ss[b],
                                  ssd[b]).wait()
            pltpu.make_async_copy(dstx.at[pl.ds(base, CHUNK)], dd[b],
                                  ssd[b]).wait()
            pltpu.make_async_copy(etype.at[pl.ds(base, CHUNK)], tt[b],
                                  ssd[b]).wait()

        def wait_outs(j, b):
            base = ebase + j * CHUNK
            pltpu.make_async_copy(gb[b], gbu_out.at[pl.ds(base, CHUNK)],
                                  so[b]).wait()
            pltpu.make_async_copy(gm[b], gmod_out.at[pl.ds(base, CHUNK)],
                                  so[b]).wait()
            pltpu.make_async_copy(mm[b], md_out.at[pl.ds(base, CHUNK)],
                                  so[b]).wait()

        issue_sdt(0, 0)
        issue_sdt(1, 1)

        def echunk2(jj, carry):
            j2 = 2 * jj
            for ph in range(2):
                b = ph
                j = j2 + ph
                wait_sdt(j, b)
                @pl.when(j >= 2)
                def _():
                    wait_outs(j - 2, b)

                dh = pltpu.async_copy(hier.at[ss[b]], hh[b], sg[b])
                dm = pltpu.async_copy(hier.at[dd[b]], mm[b], sg[b])
                dh.wait()
                dm.wait()

                def vec(i, carry2):
                    sv = ss[b][pl.ds(i * 16, 16)]
                    tv = tt[b][pl.ds(i * 16, 16)]
                    gb[b][pl.ds(i * 16, 16)] = tv * N + sv
                    gm[b][pl.ds(i * 16, 16)] = tv * M + hh[b][pl.ds(i * 16,
                                                                    16)]
                    return carry2

                lax.fori_loop(0, CHUNK // 16, vec, 0)
                base = ebase + j * CHUNK
                pltpu.async_copy(gb[b], gbu_out.at[pl.ds(base, CHUNK)], so[b])
                pltpu.async_copy(gm[b], gmod_out.at[pl.ds(base, CHUNK)], so[b])
                pltpu.async_copy(mm[b], md_out.at[pl.ds(base, CHUNK)], so[b])
                pltpu.sync_copy(ones_v, degb_acc.at[dd[b]], add=True)
                pltpu.sync_copy(ones_v, degm_acc.at[mm[b]], add=True)

                @pl.when(j + 2 < NFULL)
                def _():
                    issue_sdt(j + 2, b)
            return carry

        lax.fori_loop(0, NFULL // 2, echunk2, 0)
        wait_outs(NFULL - 2, 0)
        wait_outs(NFULL - 1, 1)

        tb = ebase + NFULL * CHUNK
        pltpu.sync_copy(srcx.at[pl.ds(tb, TAIL)], s_t)
        pltpu.sync_copy(dstx.at[pl.ds(tb, TAIL)], d_t)
        pltpu.sync_copy(etype.at[pl.ds(tb, TAIL)], t_t)
        pltpu.async_copy(hier.at[s_t], hs_t, sem).wait()
        pltpu.async_copy(hier.at[d_t], md_t, sem).wait()
        sv = s_t[...]
        tv = t_t[...]
        gbu_t[...] = tv * N + sv
        gmod_t[...] = tv * M + hs_t[...]
        pltpu.sync_copy(gbu_t, gbu_out.at[pl.ds(tb, TAIL)])
        pltpu.sync_copy(gmod_t, gmod_out.at[pl.ds(tb, TAIL)])
        pltpu.sync_copy(md_t, md_out.at[pl.ds(tb, TAIL)])
        pltpu.sync_copy(ones_v.at[pl.ds(0, TAIL)], degb_acc.at[d_t], add=True)
        pltpu.sync_copy(ones_v.at[pl.ds(0, TAIL)], degm_acc.at[md_t], add=True)

        for j in range((NCHUNK_N + NW - 1) // NW):
            cid = w + NW * j

            @pl.when(cid < NCHUNK_N)
            def _():
                pltpu.sync_copy(hier.at[pl.ds(cid * CHUNK, CHUNK)], hidx_v)
                pltpu.sync_copy(ones_v, cnt_acc.at[hidx_v], add=True)

        @pl.when(w == 0)
        def _():
            pltpu.sync_copy(hier.at[pl.ds(N - NTAIL_N, NTAIL_N)], hidx_t)
            pltpu.sync_copy(ones_v.at[pl.ds(0, NTAIL_N)], cnt_acc.at[hidx_t],
                            add=True)

        plsc.subcore_barrier()

        for j in range((NZB + NS - 1) // NS):
            zid = s + NS * j

            @pl.when(zid < NZB)
            def _():
                pltpu.sync_copy(degb_acc.at[pl.ds(zid * 80, 80)], stage_n)
                pltpu.sync_copy(stage_n,
                                degb_out.at[pl.ds(c * N + zid * 80, 80)])
        pltpu.sync_copy(degm_acc.at[pl.ds(s * rm, rm)], stage_m)
        pltpu.sync_copy(stage_m, degm_out.at[pl.ds(c * M + s * rm, rm)])
        pltpu.sync_copy(cnt_acc.at[pl.ds(s * rm, rm)], stage_m)
        pltpu.sync_copy(stage_m, cnt_out.at[pl.ds(c * M + s * rm, rm)])

    return prep


def _build_agg(NN, VT, ZCH, NZ, name):
    assert ZCH * NZ == NN
    scratch = [
        pltpu.VMEM_SHARED((NN, D), jnp.float32),
        pltpu.VMEM((CHUNK,), jnp.int32),
        pltpu.VMEM((CHUNK,), jnp.int32),
        pltpu.VMEM((CHUNK,), jnp.int32),
        pltpu.VMEM((CHUNK,), jnp.int32),
        pltpu.VMEM((CHUNK, D), jnp.float32),
        pltpu.VMEM((CHUNK, D), jnp.float32),
        pltpu.VMEM((TAIL,), jnp.int32),
        pltpu.VMEM((TAIL,), jnp.int32),
        pltpu.VMEM((TAIL, D), jnp.float32),
        pltpu.VMEM((ZCH, D), jnp.float32),
        pltpu.SemaphoreType.DMA,
        pltpu.SemaphoreType.DMA,
        pltpu.SemaphoreType.DMA,
        pltpu.SemaphoreType.DMA,
        pltpu.SemaphoreType.DMA,
        pltpu.SemaphoreType.DMA,
    ]

    @functools.partial(pl.kernel,
                       out_type=jax.ShapeDtypeStruct((NC * NN, D), jnp.float32),
                       mesh=_mesh(), scratch_types=scratch, name=name)
    def agg(table, gidx, dstx, zrows, out,
            acc, gi0, gi1, di0, di1, rows0, rows1,
            gi_t, di_t, rows_t, stage, sgi0, sgi1, sdi0, sdi1, sr0, sr1):
        c = lax.axis_index("c")
        s = lax.axis_index("s")
        w = c * NS + s
        gis = (gi0, gi1)
        dis = (di0, di1)
        rows = (rows0, rows1)
        sgis = (sgi0, sgi1)
        sdis = (sdi0, sdi1)
        srs = (sr0, sr1)

        pltpu.sync_copy(zrows, stage)
        for j in range((NZ + NS - 1) // NS):
            cid = s + NS * j

            @pl.when(cid < NZ)
            def _():
                pltpu.sync_copy(stage, acc.at[pl.ds(cid * ZCH, ZCH)])
        plsc.subcore_barrier()

        ebase = w * EW

        def issue_idx(j, b):
            base = ebase + j * CHUNK
            pltpu.async_copy(gidx.at[pl.ds(base, CHUNK)], gis[b], sgis[b])
            pltpu.async_copy(dstx.at[pl.ds(base, CHUNK)], dis[b], sdis[b])

        def wait_idx(j, b):
            base = ebase + j * CHUNK
            pltpu.make_async_copy(gidx.at[pl.ds(base, CHUNK)], gis[b],
                                  sgis[b]).wait()
            pltpu.make_async_copy(dstx.at[pl.ds(base, CHUNK)], dis[b],
                                  sdis[b]).wait()

        def issue_gather(b):
            pltpu.async_copy(table.at[gis[b]], rows[b], srs[b])

        def wait_gather(b):
            pltpu.make_async_copy(table.at[gis[b]], rows[b], srs[b]).wait()

        issue_idx(0, 0)
        issue_idx(1, 1)
        wait_idx(0, 0)
        issue_gather(0)

        def body(jj, carry):
            j = 2 * jj
            for ph in range(2):
                b = ph
                nb = 1 - ph
                wait_gather(b)

                @pl.when(j + ph + 1 < NFULL)
                def _():
                    wait_idx(j + ph + 1, nb)
                    issue_gather(nb)

                pltpu.sync_copy(rows[b], acc.at[dis[b]], add=True)

                @pl.when(j + ph + 2 < NFULL)
                def _():
                    issue_idx(j + ph + 2, b)
            return carry

        lax.fori_loop(0, NFULL // 2, body, 0)

        tb = ebase + NFULL * CHUNK
        pltpu.sync_copy(gidx.at[pl.ds(tb, TAIL)], gi_t)
        pltpu.sync_copy(dstx.at[pl.ds(tb, TAIL)], di_t)
        pltpu.async_copy(table.at[gi_t], rows_t, sr0).wait()
        pltpu.sync_copy(rows_t, acc.at[di_t], add=True)

        plsc.subcore_barrier()
        for j in range((NZ + NS - 1) // NS):
            cid = s + NS * j

            @pl.when(cid < NZ)
            def _():
                pltpu.sync_copy(acc.at[pl.ds(cid * ZCH, ZCH)], stage)
                pltpu.sync_copy(stage, out.at[pl.ds(c * NN + cid * ZCH, ZCH)])

    return agg


def _build_pool():
    RC = 80
    NRC = N // RC
    ZCH = M // NS
    scratch = [
        pltpu.VMEM_SHARED((M, D), jnp.float32),
        pltpu.VMEM((RC, D), jnp.float32),
        pltpu.VMEM((RC,), jnp.int32),
        pltpu.VMEM((ZCH, D), jnp.float32),
    ]

    @functools.partial(pl.kernel,
                       out_type=jax.ShapeDtypeStruct((NC * M, D), jnp.float32),
                       mesh=_mesh(), scratch_types=scratch, name="sc_pool")
    def pool(h, hier, zrows, out, acc, rows_v, hidx_v, stage):
        c = lax.axis_index("c")
        s = lax.axis_index("s")
        w = c * NS + s

        pltpu.sync_copy(zrows, stage)
        pltpu.sync_copy(stage, acc.at[pl.ds(s * ZCH, ZCH)])
        plsc.subcore_barrier()

        for j in range((NRC + NW - 1) // NW):
            cid = w + NW * j

            @pl.when(cid < NRC)
            def _():
                base = cid * RC
                pltpu.sync_copy(h.at[pl.ds(base, RC)], rows_v)
                pltpu.sync_copy(hier.at[pl.ds(base, RC)], hidx_v)
                pltpu.sync_copy(rows_v, acc.at[hidx_v], add=True)

        plsc.subcore_barrier()
        pltpu.sync_copy(acc.at[pl.ds(s * ZCH, ZCH)], stage)
        pltpu.sync_copy(stage, out.at[pl.ds(c * M + s * ZCH, ZCH)])

    return pool


def _dot(a, b):
    return jnp.dot(a, b, preferred_element_type=jnp.float32)


def _head_body(x_ref, wi_ref, bi_ref, wrel_ref, wself_ref, b_ref,
               hr_ref, sb_ref):
    h = jnp.maximum(_dot(x_ref[...], wi_ref[...]) + bi_ref[...], 0.0)
    for r in range(R):
        hr_ref[r] = _dot(h, wrel_ref[r])
    sb_ref[...] = _dot(h, wself_ref[...]) + b_ref[...]


def _norm_h(p_ref, deg_ref, sbp_ref):
    degs = deg_ref[0, :, 0:1] + deg_ref[1, :, 0:1]
    inv = 1.0 / jnp.maximum(degs, 1.0)
    return jnp.maximum((p_ref[0] + p_ref[1]) * inv + sbp_ref[...], 0.0)


def _mid_body(p_ref, deg_ref, sbp_ref, wrel_ref, wself_ref, b_ref,
              hr_ref, sb_ref):
    h = _norm_h(p_ref, deg_ref, sbp_ref)
    for r in range(R):
        hr_ref[r] = _dot(h, wrel_ref[r])
    sb_ref[...] = _dot(h, wself_ref[...]) + b_ref[...]


def _combine_body(p_ref, deg_ref, sbp_ref, h_ref):
    h_ref[...] = _norm_h(p_ref, deg_ref, sbp_ref)


def _mod0_body(p_ref, cnt_ref, wrel_ref, wself_ref, b_ref, hr_ref, sb_ref):
    cnts = cnt_ref[0, :, 0:1] + cnt_ref[1, :, 0:1]
    pooled = (p_ref[0] + p_ref[1]) * (1.0 / jnp.maximum(cnts, 1.0))
    for r in range(R):
        hr_ref[r] = _dot(pooled, wrel_ref[r])
    sb_ref[...] = _dot(pooled, wself_ref[...]) + b_ref[...]


def _tail_body(p_ref, deg_ref, sbp_ref, wf_ref, bf_ref, out_ref):
    h = _norm_h(p_ref, deg_ref, sbp_ref)
    out_ref[...] = jnp.maximum(_dot(h, wf_ref[...]) + bf_ref[...], 0.0)


_BN = 1000


def _head_call(x, wi, bi, wrel, wself, b):
    nb = N // _BN
    return pl.pallas_call(
        _head_body,
        grid=(nb,),
        in_specs=[
            pl.BlockSpec((_BN, D), lambda i: (i, 0)),
            pl.BlockSpec((D, D), lambda i: (0, 0)),
            pl.BlockSpec((1, D), lambda i: (0, 0)),
            pl.BlockSpec((R, D, D), lambda i: (0, 0, 0)),
            pl.BlockSpec((D, D), lambda i: (0, 0)),
            pl.BlockSpec((1, D), lambda i: (0, 0)),
        ],
        out_specs=[
            pl.BlockSpec((R, _BN, D), lambda i: (0, i, 0)),
            pl.BlockSpec((_BN, D), lambda i: (i, 0)),
        ],
        out_shape=[
            jax.ShapeDtypeStruct((R, N, D), jnp.float32),
            jax.ShapeDtypeStruct((N, D), jnp.float32),
        ],
    )(x, wi, bi.reshape(1, D), wrel, wself, b.reshape(1, D))


def _mid_call(part, deg2, sbp, wrel, wself, b):
    nb = N // _BN
    return pl.pallas_call(
        _mid_body,
        grid=(nb,),
        in_specs=[
            pl.BlockSpec((NC, _BN, D), lambda i: (0, i, 0)),
            pl.BlockSpec((NC, _BN, 1), lambda i: (0, i, 0)),
            pl.BlockSpec((_BN, D), lambda i: (i, 0)),
            pl.BlockSpec((R, D, D), lambda i: (0, 0, 0)),
            pl.BlockSpec((D, D), lambda i: (0, 0)),
            pl.BlockSpec((1, D), lambda i: (0, 0)),
        ],
        out_specs=[
            pl.BlockSpec((R, _BN, D), lambda i: (0, i, 0)),
            pl.BlockSpec((_BN, D), lambda i: (i, 0)),
        ],
        out_shape=[
            jax.ShapeDtypeStruct((R, N, D), jnp.float32),
            jax.ShapeDtypeStruct((N, D), jnp.float32),
        ],
    )(part.reshape(NC, N, D), deg2.reshape(NC, N, 1), sbp,
      wrel, wself, b.reshape(1, D))


def _combine_call(part, deg2, sbp):
    nb = N // _BN
    return pl.pallas_call(
        _combine_body,
        grid=(nb,),
        in_specs=[
            pl.BlockSpec((NC, _BN, D), lambda i: (0, i, 0)),
            pl.BlockSpec((NC, _BN, 1), lambda i: (0, i, 0)),
            pl.BlockSpec((_BN, D), lambda i: (i, 0)),
        ],
        out_specs=pl.BlockSpec((_BN, D), lambda i: (i, 0)),
        out_shape=jax.ShapeDtypeStruct((N, D), jnp.float32),
    )(part.reshape(NC, N, D), deg2.reshape(NC, N, 1), sbp)


def _mod0_call(pool2, cnt2, wrel, wself, b):
    return pl.pallas_call(
        _mod0_body,
        out_shape=[
            jax.ShapeDtypeStruct((R, M, D), jnp.float32),
            jax.ShapeDtypeStruct((M, D), jnp.float32),
        ],
    )(pool2.reshape(NC, M, D), cnt2.reshape(NC, M, 1),
      wrel, wself, b.reshape(1, D))


def _modmid_call(part, deg2, sbp, wrel, wself, b):
    return pl.pallas_call(
        _mid_body,
        out_shape=[
            jax.ShapeDtypeStruct((R, M, D), jnp.float32),
            jax.ShapeDtypeStruct((M, D), jnp.float32),
        ],
    )(part.reshape(NC, M, D), deg2.reshape(NC, M, 1), sbp,
      wrel, wself, b.reshape(1, D))


def _tail_call(part, deg2, sbp, wf, bf):
    return pl.pallas_call(
        _tail_body,
        out_shape=jax.ShapeDtypeStruct((M, D), jnp.float32),
    )(part.reshape(NC, M, D), deg2.reshape(NC, M, 1), sbp,
      wf, bf.reshape(1, D))


_prep = _build_prep()
_agg_n = _build_agg(N, R * N, 80, 125, "sc_agg_n")
_agg_m = _build_agg(M, R * M, 64, 16, "sc_agg_m")
_pool = _build_pool()


def kernel(x, edge_index, edge_type, hierarchy,
           W_init, b_init, W_rel_bu, W_self_bu, b_bu,
           W_rel_mod, W_self_mod, b_mod, W_fin, b_fin):
    srcx = edge_index[0]
    dstx = edge_index[1]
    ones_h = jnp.ones((CHUNK,), jnp.float32)
    zn_h = jnp.zeros((80,), jnp.float32)
    zm_h = jnp.zeros((M // NS,), jnp.float32)
    z80d = jnp.zeros((80, D), jnp.float32)
    z64d = jnp.zeros((64, D), jnp.float32)

    gbu, gmod, mdst, degb2, degm2, cnt2 = _prep(srcx, dstx, edge_type,
                                                hierarchy, ones_h, zn_h, zm_h)

    hr, sb = _head_call(x, W_init, b_init, W_rel_bu[0], W_self_bu[0], b_bu[0])
    part = _agg_n(hr.reshape(R * N, D), gbu, dstx, z80d)
    hr, sb = _mid_call(part, degb2, sb, W_rel_bu[1], W_self_bu[1], b_bu[1])
    part = _agg_n(hr.reshape(R * N, D), gbu, dstx, z80d)
    h2 = _combine_call(part, degb2, sb)

    pool2 = _pool(h2, hierarchy, z64d)
    hrm, sbm = _mod0_call(pool2, cnt2, W_rel_mod[0], W_self_mod[0], b_mod[0])
    mp = _agg_m(hrm.reshape(R * M, D), gmod, mdst, z64d)
    hrm, sbm = _modmid_call(mp, degm2, sbm, W_rel_mod[1], W_self_mod[1],
                            b_mod[1])
    mp = _agg_m(hrm.reshape(R * M, D), gmod, mdst, z64d)

    return _tail_call(mp, degm2, sbm, W_fin, b_fin)


# speedup vs baseline: 41.0392x; 1.0084x over previous
import functools

import jax
import jax.numpy as jnp
from jax import lax
from jax.experimental import pallas as pl
from jax.experimental.pallas import tpu as pltpu
from jax.experimental.pallas import tpu_sc as plsc

N = 10000
E = 320000
D = 128
R = 4
M = 1024

NC = 2
NS = 16
NW = NC * NS
EW = E // NW
CHUNK = 128
NFULL = EW // CHUNK
TAIL = EW - NFULL * CHUNK
NCHUNK_N = N // CHUNK
NTAIL_N = N - NCHUNK_N * CHUNK


def _mesh():
    return plsc.VectorSubcoreMesh(core_axis_name="c", subcore_axis_name="s")


def _build_prep():
    out_type = (
        jax.ShapeDtypeStruct((E,), jnp.int32),
        jax.ShapeDtypeStruct((E,), jnp.int32),
        jax.ShapeDtypeStruct((E,), jnp.int32),
        jax.ShapeDtypeStruct((NC * N,), jnp.float32),
        jax.ShapeDtypeStruct((NC * M,), jnp.float32),
        jax.ShapeDtypeStruct((NC * M,), jnp.float32),
    )
    scratch = [
        pltpu.VMEM((CHUNK,), jnp.int32),
        pltpu.VMEM((CHUNK,), jnp.int32),
        pltpu.VMEM((TAIL,), jnp.int32),
        pltpu.VMEM((CHUNK,), jnp.int32),
        pltpu.VMEM((CHUNK,), jnp.int32),
        pltpu.VMEM((CHUNK,), jnp.int32),
        pltpu.VMEM((CHUNK,), jnp.int32),
        pltpu.VMEM((CHUNK,), jnp.int32),
        pltpu.VMEM((CHUNK,), jnp.int32),
        pltpu.VMEM((CHUNK,), jnp.int32),
        pltpu.VMEM((CHUNK,), jnp.int32),
        pltpu.VMEM((CHUNK,), jnp.int32),
        pltpu.VMEM((CHUNK,), jnp.int32),
        pltpu.VMEM((CHUNK,), jnp.int32),
        pltpu.VMEM((CHUNK,), jnp.int32),
        pltpu.VMEM((TAIL,), jnp.int32),
        pltpu.VMEM((TAIL,), jnp.int32),
        pltpu.VMEM((TAIL,), jnp.int32),
        pltpu.VMEM((TAIL,), jnp.int32),
        pltpu.VMEM((TAIL,), jnp.int32),
        pltpu.VMEM((TAIL,), jnp.int32),
        pltpu.VMEM((CHUNK,), jnp.int32),
        pltpu.VMEM((NTAIL_N,), jnp.int32),
        pltpu.VMEM((CHUNK,), jnp.float32),
        pltpu.VMEM((80,), jnp.float32),
        pltpu.VMEM((M // NS,), jnp.float32),
        pltpu.VMEM_SHARED((N,), jnp.float32),
        pltpu.VMEM_SHARED((M,), jnp.float32),
        pltpu.VMEM_SHARED((M,), jnp.float32),
        pltpu.SemaphoreType.DMA,
        pltpu.SemaphoreType.DMA,
        pltpu.SemaphoreType.DMA,
        pltpu.SemaphoreType.DMA,
        pltpu.SemaphoreType.DMA,
        pltpu.SemaphoreType.DMA,
        pltpu.SemaphoreType.DMA,
    ]

    @functools.partial(pl.kernel, out_type=out_type, mesh=_mesh(),
                       scratch_types=scratch, name="sc_prep")
    def prep(srcx, dstx, etype, hier, ones_h, zn_h, zm_h,
             gbu_out, gmod_out, md_out, degb_out, degm_out, cnt_out,
             hs0, hs1, hs_t, s0, s1, d0, d1, t0, t1,
             gbu0, gbu1, gmod0, gmod1, md0, md1,
             s_t, d_t, t_t, gbu_t, gmod_t, md_t, hidx_v, hidx_t,
             ones_v, stage_n, stage_m, degb_acc, degm_acc, cnt_acc,
             sem, ssd0, ssd1, sg0, sg1, so0, so1):
        c = lax.axis_index("c")
        s = lax.axis_index("s")
        w = c * NS + s
        rm = M // NS
        NZB = N // 80

        pltpu.sync_copy(zn_h, stage_n)
        for j in range((NZB + NS - 1) // NS):
            zid = s + NS * j

            @pl.when(zid < NZB)
            def _():
                pltpu.sync_copy(stage_n, degb_acc.at[pl.ds(zid * 80, 80)])
        pltpu.sync_copy(zm_h, stage_m)
        pltpu.sync_copy(stage_m, degm_acc.at[pl.ds(s * rm, rm)])
        pltpu.sync_copy(stage_m, cnt_acc.at[pl.ds(s * rm, rm)])
        plsc.subcore_barrier()

        pltpu.sync_copy(ones_h, ones_v)

        ebase = w * EW
        ss = (s0, s1)
        dd = (d0, d1)
        tt = (t0, t1)
        hh = (hs0, hs1)
        mm = (md0, md1)
        gb = (gbu0, gbu1)
        gm = (gmod0, gmod1)
        ssd = (ssd0, ssd1)
        sg = (sg0, sg1)
        so = (so0, so1)

        def issue_sdt(j, b):
            base = ebase + j * CHUNK
            pltpu.async_copy(srcx.at[pl.ds(base, CHUNK)], ss[b], ssd[b])
            pltpu.async_copy(dstx.at[pl.ds(base, CHUNK)], dd[b], ssd[b])
            pltpu.async_copy(etype.at[pl.ds(base, CHUNK)], tt[b], ssd[b])

        def wait_sdt(j, b):
            base = ebase + j * CHUNK
            pltpu.make_async_copy(srcx.at[pl.ds(base, CHUNK)], ss[b],
                                  ssd[b]).wait()
            pltpu.make_async_copy(dstx.at[pl.ds(base, CHUNK)], dd[b],
                                  ssd[b]).wait()
            pltpu.make_async_copy(etype.at[pl.ds(base, CHUNK)], tt[b],
                                  ssd[b]).wait()

        def wait_outs(j, b):
            base = ebase + j * CHUNK
            pltpu.make_async_copy(gb[b], gbu_out.at[pl.ds(base, CHUNK)],
                                  so[b]).wait()
            pltpu.make_async_copy(gm[b], gmod_out.at[pl.ds(base, CHUNK)],
                                  so[b]).wait()
            pltpu.make_async_copy(mm[b], md_out.at[pl.ds(base, CHUNK)],
                                  so[b]).wait()

        issue_sdt(0, 0)
        issue_sdt(1, 1)

        def echunk2(jj, carry):
            j2 = 2 * jj
            for ph in range(2):
                b = ph
                j = j2 + ph
                wait_sdt(j, b)
                @pl.when(j >= 2)
                def _():
                    wait_outs(j - 2, b)

                dh = pltpu.async_copy(hier.at[ss[b]], hh[b], sg[b])
                dm = pltpu.async_copy(hier.at[dd[b]], mm[b], sg[b])
                dh.wait()
                dm.wait()

                def vec(i, carry2):
                    sv = ss[b][pl.ds(i * 16, 16)]
                    tv = tt[b][pl.ds(i * 16, 16)]
                    gb[b][pl.ds(i * 16, 16)] = tv * N + sv
                    gm[b][pl.ds(i * 16, 16)] = tv * M + hh[b][pl.ds(i * 16,
                                                                    16)]
                    return carry2

                lax.fori_loop(0, CHUNK // 16, vec, 0)
                base = ebase + j * CHUNK
                pltpu.async_copy(gb[b], gbu_out.at[pl.ds(base, CHUNK)], so[b])
                pltpu.async_copy(gm[b], gmod_out.at[pl.ds(base, CHUNK)], so[b])
                pltpu.async_copy(mm[b], md_out.at[pl.ds(base, CHUNK)], so[b])
                pltpu.sync_copy(ones_v, degb_acc.at[dd[b]], add=True)
                pltpu.sync_copy(ones_v, degm_acc.at[mm[b]], add=True)

                @pl.when(j + 2 < NFULL)
                def _():
                    issue_sdt(j + 2, b)
            return carry

        lax.fori_loop(0, NFULL // 2, echunk2, 0)
        wait_outs(NFULL - 2, 0)
        wait_outs(NFULL - 1, 1)

        tb = ebase + NFULL * CHUNK
        pltpu.sync_copy(srcx.at[pl.ds(tb, TAIL)], s_t)
        pltpu.sync_copy(dstx.at[pl.ds(tb, TAIL)], d_t)
        pltpu.sync_copy(etype.at[pl.ds(tb, TAIL)], t_t)
        pltpu.async_copy(hier.at[s_t], hs_t, sem).wait()
        pltpu.async_copy(hier.at[d_t], md_t, sem).wait()
        sv = s_t[...]
        tv = t_t[...]
        gbu_t[...] = tv * N + sv
        gmod_t[...] = tv * M + hs_t[...]
        pltpu.sync_copy(gbu_t, gbu_out.at[pl.ds(tb, TAIL)])
        pltpu.sync_copy(gmod_t, gmod_out.at[pl.ds(tb, TAIL)])
        pltpu.sync_copy(md_t, md_out.at[pl.ds(tb, TAIL)])
        pltpu.sync_copy(ones_v.at[pl.ds(0, TAIL)], degb_acc.at[d_t], add=True)
        pltpu.sync_copy(ones_v.at[pl.ds(0, TAIL)], degm_acc.at[md_t], add=True)

        for j in range((NCHUNK_N + NW - 1) // NW):
            cid = w + NW * j

            @pl.when(cid < NCHUNK_N)
            def _():
                pltpu.sync_copy(hier.at[pl.ds(cid * CHUNK, CHUNK)], hidx_v)
                pltpu.sync_copy(ones_v, cnt_acc.at[hidx_v], add=True)

        @pl.when(w == 0)
        def _():
            pltpu.sync_copy(hier.at[pl.ds(N - NTAIL_N, NTAIL_N)], hidx_t)
            pltpu.sync_copy(ones_v.at[pl.ds(0, NTAIL_N)], cnt_acc.at[hidx_t],
                            add=True)

        plsc.subcore_barrier()

        for j in range((NZB + NS - 1) // NS):
            zid = s + NS * j

            @pl.when(zid < NZB)
            def _():
                pltpu.sync_copy(degb_acc.at[pl.ds(zid * 80, 80)], stage_n)
                pltpu.sync_copy(stage_n,
                                degb_out.at[pl.ds(c * N + zid * 80, 80)])
        pltpu.sync_copy(degm_acc.at[pl.ds(s * rm, rm)], stage_m)
        pltpu.sync_copy(stage_m, degm_out.at[pl.ds(c * M + s * rm, rm)])
        pltpu.sync_copy(cnt_acc.at[pl.ds(s * rm, rm)], stage_m)
        pltpu.sync_copy(stage_m, cnt_out.at[pl.ds(c * M + s * rm, rm)])

    return prep


def _build_agg(NN, VT, ZCH, NZ, name):
    assert ZCH * NZ == NN
    scratch = [
        pltpu.VMEM_SHARED((NN, D), jnp.float32),
        pltpu.VMEM((CHUNK,), jnp.int32),
        pltpu.VMEM((CHUNK,), jnp.int32),
        pltpu.VMEM((CHUNK,), jnp.int32),
        pltpu.VMEM((CHUNK,), jnp.int32),
        pltpu.VMEM((CHUNK, D), jnp.float32),
        pltpu.VMEM((CHUNK, D), jnp.float32),
        pltpu.VMEM((TAIL,), jnp.int32),
        pltpu.VMEM((TAIL,), jnp.int32),
        pltpu.VMEM((TAIL, D), jnp.float32),
        pltpu.VMEM((ZCH, D), jnp.float32),
        pltpu.SemaphoreType.DMA,
        pltpu.SemaphoreType.DMA,
        pltpu.SemaphoreType.DMA,
        pltpu.SemaphoreType.DMA,
        pltpu.SemaphoreType.DMA,
        pltpu.SemaphoreType.DMA,
    ]

    @functools.partial(pl.kernel,
                       out_type=jax.ShapeDtypeStruct((NC * NN, D), jnp.float32),
                       mesh=_mesh(), scratch_types=scratch, name=name)
    def agg(table, gidx, dstx, zrows, out,
            acc, gi0, gi1, di0, di1, rows0, rows1,
            gi_t, di_t, rows_t, stage, sgi0, sgi1, sdi0, sdi1, sr0, sr1):
        c = lax.axis_index("c")
        s = lax.axis_index("s")
        w = c * NS + s
        gis = (gi0, gi1)
        dis = (di0, di1)
        rows = (rows0, rows1)
        sgis = (sgi0, sgi1)
        sdis = (sdi0, sdi1)
        srs = (sr0, sr1)

        pltpu.sync_copy(zrows, stage)
        for j in range((NZ + NS - 1) // NS):
            cid = s + NS * j

            @pl.when(cid < NZ)
            def _():
                pltpu.sync_copy(stage, acc.at[pl.ds(cid * ZCH, ZCH)])
        plsc.subcore_barrier()

        ebase = w * EW

        def issue_idx(j, b):
            base = ebase + j * CHUNK
            pltpu.async_copy(gidx.at[pl.ds(base, CHUNK)], gis[b], sgis[b])
            pltpu.async_copy(dstx.at[pl.ds(base, CHUNK)], dis[b], sdis[b])

        def wait_idx(j, b):
            base = ebase + j * CHUNK
            pltpu.make_async_copy(gidx.at[pl.ds(base, CHUNK)], gis[b],
                                  sgis[b]).wait()
            pltpu.make_async_copy(dstx.at[pl.ds(base, CHUNK)], dis[b],
                                  sdis[b]).wait()

        def issue_gather(b):
            pltpu.async_copy(table.at[gis[b]], rows[b], srs[b])

        def wait_gather(b):
            pltpu.make_async_copy(table.at[gis[b]], rows[b], srs[b]).wait()

        issue_idx(0, 0)
        issue_idx(1, 1)
        wait_idx(0, 0)
        issue_gather(0)

        def body(jj, carry):
            j = 2 * jj
            for ph in range(2):
                b = ph
                nb = 1 - ph
                wait_gather(b)

                @pl.when(j + ph + 1 < NFULL)
                def _():
                    wait_idx(j + ph + 1, nb)
                    issue_gather(nb)

                pltpu.sync_copy(rows[b], acc.at[dis[b]], add=True)

                @pl.when(j + ph + 2 < NFULL)
                def _():
                    issue_idx(j + ph + 2, b)
            return carry

        lax.fori_loop(0, NFULL // 2, body, 0)

        tb = ebase + NFULL * CHUNK
        pltpu.sync_copy(gidx.at[pl.ds(tb, TAIL)], gi_t)
        pltpu.sync_copy(dstx.at[pl.ds(tb, TAIL)], di_t)
        pltpu.async_copy(table.at[gi_t], rows_t, sr0).wait()
        pltpu.sync_copy(rows_t, acc.at[di_t], add=True)

        plsc.subcore_barrier()
        for j in range((NZ + NS - 1) // NS):
            cid = s + NS * j

            @pl.when(cid < NZ)
            def _():
                pltpu.sync_copy(acc.at[pl.ds(cid * ZCH, ZCH)], stage)
                pltpu.sync_copy(stage, out.at[pl.ds(c * NN + cid * ZCH, ZCH)])

    return agg


def _build_pool():
    RC = 80
    NRC = N // RC
    ZCH = M // NS
    scratch = [
        pltpu.VMEM_SHARED((M, D), jnp.float32),
        pltpu.VMEM((RC, D), jnp.float32),
        pltpu.VMEM((RC,), jnp.int32),
        pltpu.VMEM((ZCH, D), jnp.float32),
    ]

    @functools.partial(pl.kernel,
                       out_type=jax.ShapeDtypeStruct((NC * M, D), jnp.float32),
                       mesh=_mesh(), scratch_types=scratch, name="sc_pool")
    def pool(h, hier, zrows, out, acc, rows_v, hidx_v, stage):
        c = lax.axis_index("c")
        s = lax.axis_index("s")
        w = c * NS + s

        pltpu.sync_copy(zrows, stage)
        pltpu.sync_copy(stage, acc.at[pl.ds(s * ZCH, ZCH)])
        plsc.subcore_barrier()

        for j in range((NRC + NW - 1) // NW):
            cid = w + NW * j

            @pl.when(cid < NRC)
            def _():
                base = cid * RC
                pltpu.sync_copy(h.at[pl.ds(base, RC)], rows_v)
                pltpu.sync_copy(hier.at[pl.ds(base, RC)], hidx_v)
                pltpu.sync_copy(rows_v, acc.at[hidx_v], add=True)

        plsc.subcore_barrier()
        pltpu.sync_copy(acc.at[pl.ds(s * ZCH, ZCH)], stage)
        pltpu.sync_copy(stage, out.at[pl.ds(c * M + s * ZCH, ZCH)])

    return pool


def _dot(a, b):
    return jnp.dot(a, b, preferred_element_type=jnp.float32)


def _head_body(x_ref, wi_ref, bi_ref, wrel_ref, wself_ref, b_ref,
               hr_ref, sb_ref):
    h = jnp.maximum(_dot(x_ref[...], wi_ref[...]) + bi_ref[...], 0.0)
    for r in range(R):
        hr_ref[r] = _dot(h, wrel_ref[r])
    sb_ref[...] = _dot(h, wself_ref[...]) + b_ref[...]


def _norm_h(p_ref, deg_ref, sbp_ref):
    degs = deg_ref[0, :, 0:1] + deg_ref[1, :, 0:1]
    inv = 1.0 / jnp.maximum(degs, 1.0)
    return jnp.maximum((p_ref[0] + p_ref[1]) * inv + sbp_ref[...], 0.0)


def _mid_body(p_ref, deg_ref, sbp_ref, wrel_ref, wself_ref, b_ref,
              hr_ref, sb_ref):
    h = _norm_h(p_ref, deg_ref, sbp_ref)
    for r in range(R):
        hr_ref[r] = _dot(h, wrel_ref[r])
    sb_ref[...] = _dot(h, wself_ref[...]) + b_ref[...]


def _combine_body(p_ref, deg_ref, sbp_ref, h_ref):
    h_ref[...] = _norm_h(p_ref, deg_ref, sbp_ref)


def _mod0_body(p_ref, cnt_ref, wrel_ref, wself_ref, b_ref, hr_ref, sb_ref):
    cnts = cnt_ref[0, :, 0:1] + cnt_ref[1, :, 0:1]
    pooled = (p_ref[0] + p_ref[1]) * (1.0 / jnp.maximum(cnts, 1.0))
    for r in range(R):
        hr_ref[r] = _dot(pooled, wrel_ref[r])
    sb_ref[...] = _dot(pooled, wself_ref[...]) + b_ref[...]


def _tail_body(p_ref, deg_ref, sbp_ref, wf_ref, bf_ref, out_ref):
    h = _norm_h(p_ref, deg_ref, sbp_ref)
    out_ref[...] = jnp.maximum(_dot(h, wf_ref[...]) + bf_ref[...], 0.0)


_BN = 1000


def _head_call(x, wi, bi, wrel, wself, b):
    nb = N // _BN
    return pl.pallas_call(
        _head_body,
        grid=(nb,),
        in_specs=[
            pl.BlockSpec((_BN, D), lambda i: (i, 0)),
            pl.BlockSpec((D, D), lambda i: (0, 0)),
            pl.BlockSpec((1, D), lambda i: (0, 0)),
            pl.BlockSpec((R, D, D), lambda i: (0, 0, 0)),
            pl.BlockSpec((D, D), lambda i: (0, 0)),
            pl.BlockSpec((1, D), lambda i: (0, 0)),
        ],
        out_specs=[
            pl.BlockSpec((R, _BN, D), lambda i: (0, i, 0)),
            pl.BlockSpec((_BN, D), lambda i: (i, 0)),
        ],
        out_shape=[
            jax.ShapeDtypeStruct((R, N, D), jnp.float32),
            jax.ShapeDtypeStruct((N, D), jnp.float32),
        ],
    )(x, wi, bi.reshape(1, D), wrel, wself, b.reshape(1, D))


def _mid_call(part, deg2, sbp, wrel, wself, b):
    nb = N // _BN
    return pl.pallas_call(
        _mid_body,
        grid=(nb,),
        in_specs=[
            pl.BlockSpec((NC, _BN, D), lambda i: (0, i, 0)),
            pl.BlockSpec((NC, _BN, 1), lambda i: (0, i, 0)),
            pl.BlockSpec((_BN, D), lambda i: (i, 0)),
            pl.BlockSpec((R, D, D), lambda i: (0, 0, 0)),
            pl.BlockSpec((D, D), lambda i: (0, 0)),
            pl.BlockSpec((1, D), lambda i: (0, 0)),
        ],
        out_specs=[
            pl.BlockSpec((R, _BN, D), lambda i: (0, i, 0)),
            pl.BlockSpec((_BN, D), lambda i: (i, 0)),
        ],
        out_shape=[
            jax.ShapeDtypeStruct((R, N, D), jnp.float32),
            jax.ShapeDtypeStruct((N, D), jnp.float32),
        ],
    )(part.reshape(NC, N, D), deg2.reshape(NC, N, 1), sbp,
      wrel, wself, b.reshape(1, D))


def _combine_call(part, deg2, sbp):
    nb = N // _BN
    return pl.pallas_call(
        _combine_body,
        grid=(nb,),
        in_specs=[
            pl.BlockSpec((NC, _BN, D), lambda i: (0, i, 0)),
            pl.BlockSpec((NC, _BN, 1), lambda i: (0, i, 0)),
            pl.BlockSpec((_BN, D), lambda i: (i, 0)),
        ],
        out_specs=pl.BlockSpec((_BN, D), lambda i: (i, 0)),
        out_shape=jax.ShapeDtypeStruct((N, D), jnp.float32),
    )(part.reshape(NC, N, D), deg2.reshape(NC, N, 1), sbp)


def _mod0_call(pool2, cnt2, wrel, wself, b):
    return pl.pallas_call(
        _mod0_body,
        out_shape=[
            jax.ShapeDtypeStruct((R, M, D), jnp.float32),
            jax.ShapeDtypeStruct((M, D), jnp.float32),
        ],
    )(pool2.reshape(NC, M, D), cnt2.reshape(NC, M, 1),
      wrel, wself, b.reshape(1, D))


def _modmid_call(part, deg2, sbp, wrel, wself, b):
    return pl.pallas_call(
        _mid_body,
        out_shape=[
            jax.ShapeDtypeStruct((R, M, D), jnp.float32),
            jax.ShapeDtypeStruct((M, D), jnp.float32),
        ],
    )(part.reshape(NC, M, D), deg2.reshape(NC, M, 1), sbp,
      wrel, wself, b.reshape(1, D))


def _tail_call(part, deg2, sbp, wf, bf):
    return pl.pallas_call(
        _tail_body,
        out_shape=jax.ShapeDtypeStruct((M, D), jnp.float32),
    )(part.reshape(NC, M, D), deg2.reshape(NC, M, 1), sbp,
      wf, bf.reshape(1, D))


_prep = _build_prep()
_agg_n = _build_agg(N, R * N, 80, 125, "sc_agg_n")
_agg_m = _build_agg(M, R * M, 64, 16, "sc_agg_m")
_pool = _build_pool()


def kernel(x, edge_index, edge_type, hierarchy,
           W_init, b_init, W_rel_bu, W_self_bu, b_bu,
           W_rel_mod, W_self_mod, b_mod, W_fin, b_fin):
    srcx = edge_index[0]
    dstx = edge_index[1]
    ones_h = jnp.ones((CHUNK,), jnp.float32)
    zn_h = jnp.zeros((80,), jnp.float32)
    zm_h = jnp.zeros((M // NS,), jnp.float32)
    z80d = jnp.zeros((80, D), jnp.float32)
    z64d = jnp.zeros((64, D), jnp.float32)

    gbu, gmod, mdst, degb2, degm2, cnt2 = _prep(srcx, dstx, edge_type,
                                                hierarchy, ones_h, zn_h, zm_h)

    hr, sb = _head_call(x, W_init, b_init, W_rel_bu[0], W_self_bu[0], b_bu[0])
    part = _agg_n(hr.reshape(R * N, D), gbu, dstx, z80d)
    hr, sb = _mid_call(part, degb2, sb, W_rel_bu[1], W_self_bu[1], b_bu[1])
    part = _agg_n(hr.reshape(R * N, D), gbu, dstx, z80d)
    h2 = _combine_call(part, degb2, sb)

    pool2 = _pool(h2, hierarchy, z64d)
    hrm, sbm = _mod0_call(pool2, cnt2, W_rel_mod[0], W_self_mod[0], b_mod[0])
    mp = _agg_m(hrm.reshape(R * M, D), gmod, mdst, z64d)
    hrm, sbm = _modmid_call(mp, degm2, sbm, W_rel_mod[1], W_self_mod[1],
                            b_mod[1])
    mp = _agg_m(hrm.reshape(R * M, D), gmod, mdst, z64d)

    return _tail_call(mp, degm2, sbm, W_fin, b_fin)
